# baseline (speedup 1.0000x reference)
import functools

import jax
import jax.numpy as jnp
from jax import lax
from jax.experimental import pallas as pl
from jax.experimental.pallas import tpu as pltpu
from jax.experimental.pallas import tpu_sc as plsc

NC = 2
NS = 16
NW = NC * NS
CHUNK = 128
NBUF = 2
NISLOT = 4
N_BLK = 1024


def _hist_body(idx_hbm, out_hbm, idx_v, hist_v):
    cid = lax.axis_index("c")
    sid = lax.axis_index("s")
    wid = sid * NC + cid
    hc = idx_v.shape[0]
    pltpu.sync_copy(idx_hbm.at[pl.ds(wid * hc, hc)], idx_v)
    nvec = hist_v.shape[0] // 16

    def zero_body(i, c):
        hist_v[pl.ds(i * 16, 16)] = jnp.zeros((16,), jnp.float32)
        return c

    lax.fori_loop(0, nvec, zero_body, 0)
    ones = jnp.ones((16,), jnp.float32)
    nchunk = idx_v.shape[0]

    def chunk_body(j, c):
        for k in range(CHUNK // 16):
            idx = idx_v[j, 2, pl.ds(k * 16, 16)]
            plsc.addupdate_scatter(hist_v, [idx], ones)
        return c

    lax.fori_loop(0, nchunk, chunk_body, 0)
    pltpu.sync_copy(hist_v, out_hbm.at[wid])


def _sc_hist(idx4, n_hist):
    hc = idx4.shape[0] // NW
    mesh = plsc.VectorSubcoreMesh(core_axis_name="c", subcore_axis_name="s")
    f = pl.kernel(
        _hist_body,
        out_type=jax.ShapeDtypeStruct((NW, n_hist), jnp.float32),
        mesh=mesh,
        compiler_params=pltpu.CompilerParams(needs_layout_passes=False),
        scratch_types=[
            pltpu.VMEM((hc, 4, CHUNK), jnp.int32),
            pltpu.VMEM((n_hist,), jnp.float32),
        ],
    )
    return f(idx4)


def _agg_body(n0, n1, y_hbm, idx_hbm, out_hbm,
              ib, rows_v, zsrc, acc_sh, isem, gsem, ssem):
    cid = lax.axis_index("c")
    sid = lax.axis_index("s")
    n_acc = acc_sh.shape[0]
    d = acc_sh.shape[1]
    nis = ib.shape[0]
    nchunk = jnp.where(cid == 0, n0, n1)
    base = jnp.where(cid == 0, sid * n0, NS * n0 + sid * n1)

    pltpu.async_copy(idx_hbm.at[base], ib.at[0], isem.at[0])
    pltpu.async_copy(idx_hbm.at[base + 1], ib.at[1], isem.at[1])

    zr = zsrc.shape[0]

    def zstore(r, c):
        for k in range(CHUNK // 16):
            zsrc[r, pl.ds(k * 16, 16)] = jnp.zeros((16,), jnp.float32)
        return c

    lax.fori_loop(0, zr, zstore, 0)
    zrow = n_acc // NS
    for t in range(zrow // zr):
        pltpu.sync_copy(zsrc, acc_sh.at[pl.ds(sid * zrow + t * zr, zr)])
    plsc.subcore_barrier()

    pltpu.make_async_copy(idx_hbm.at[base], ib.at[0], isem.at[0]).wait()
    pltpu.async_copy(y_hbm.at[ib.at[0, cid]], rows_v.at[0], gsem.at[0])

    def chunk_body(j, c):
        @pl.when(j >= 1)
        def _():
            b = lax.rem(j - 1, NBUF)
            i = lax.rem(j - 1, nis)
            pltpu.make_async_copy(rows_v.at[b], acc_sh.at[ib.at[i, 2]],
                                  ssem.at[b]).wait()

        @pl.when(j + 2 < nchunk)
        def _():
            i = lax.rem(j + 2, nis)
            pltpu.async_copy(idx_hbm.at[base + j + 2], ib.at[i], isem.at[i])

        @pl.when(j + 1 < nchunk)
        def _():
            b = lax.rem(j + 1, NBUF)
            i = lax.rem(j + 1, nis)
            pltpu.make_async_copy(idx_hbm.at[base + j + 1], ib.at[i],
                                  isem.at[i]).wait()
            pltpu.async_copy(y_hbm.at[ib.at[i, cid]], rows_v.at[b], gsem.at[b])

        b = lax.rem(j, NBUF)
        i = lax.rem(j, nis)
        pltpu.make_async_copy(y_hbm.at[ib.at[i, cid]], rows_v.at[b],
                              gsem.at[b]).wait()
        pltpu.async_copy(rows_v.at[b], acc_sh.at[ib.at[i, 2]], ssem.at[b],
                         add=True)
        return c

    lax.fori_loop(0, nchunk, chunk_body, 0)
    j = nchunk - 1
    pltpu.make_async_copy(rows_v.at[lax.rem(j, NBUF)],
                          acc_sh.at[ib.at[lax.rem(j, nis), 2]],
                          ssem.at[lax.rem(j, NBUF)]).wait()
    plsc.subcore_barrier()
    orow = n_acc // NS
    pltpu.sync_copy(acc_sh.at[pl.ds(sid * orow, orow)],
                    out_hbm.at[cid, pl.ds(sid * orow, orow)])


def _sc_aggregate(y, idx4, n0, n1, n_acc):
    d = y.shape[1]
    mesh = plsc.VectorSubcoreMesh(core_axis_name="c", subcore_axis_name="s")
    f = pl.kernel(
        functools.partial(_agg_body, n0, n1),
        out_type=jax.ShapeDtypeStruct((NC, n_acc, d), jnp.float32),
        mesh=mesh,
        scratch_types=[
            pltpu.VMEM((NISLOT, 4, CHUNK), jnp.int32),
            pltpu.VMEM((NBUF, CHUNK, d), jnp.float32),
            pltpu.VMEM((64, d), jnp.float32),
            pltpu.VMEM_SHARED((n_acc, d), jnp.float32),
            pltpu.SemaphoreType.DMA((NISLOT,)),
            pltpu.SemaphoreType.DMA((NBUF,)),
            pltpu.SemaphoreType.DMA((NBUF,)),
        ],
    )
    return f(y, idx4)


def _prep_tc(hist, x, w1):
    n, d = x.shape
    g = pl.cdiv(n, N_BLK)

    def body(hist_ref, x_ref, w_ref, y_ref, y1_ref, dis_ref):
        deg = jnp.sum(hist_ref[...], axis=0) + 1.0
        dis = lax.rsqrt(deg)
        y = jnp.dot(x_ref[...], w_ref[...],
                    preferred_element_type=jnp.float32) * dis[:, None]
        y_ref[...] = y
        y1_ref[...] = y
        dis_ref[...] = dis[:, None]

    return pl.pallas_call(
        body,
        grid=(g,),
        in_specs=[
            pl.BlockSpec((NW, N_BLK), lambda i: (0, i)),
            pl.BlockSpec((N_BLK, d), lambda i: (i, 0)),
            pl.BlockSpec((d, d), lambda i: (0, 0)),
        ],
        out_specs=[
            pl.BlockSpec((N_BLK, d), lambda i: (i, 0)),
            pl.BlockSpec((N_BLK, d), lambda i: (i, 0)),
            pl.BlockSpec((N_BLK, 1), lambda i: (i, 0)),
        ],
        out_shape=[
            jax.ShapeDtypeStruct((n, d), jnp.float32),
            jax.ShapeDtypeStruct((n, d), jnp.float32),
            jax.ShapeDtypeStruct((n, 1), jnp.float32),
        ],
    )(hist, x, w1)


def _mid_tc(p, y, dis, b, w_next):
    n, d = y.shape
    g = pl.cdiv(n, N_BLK)

    def body(p0_ref, p1_ref, y_ref, dis_ref, b_ref, w_ref, o_ref, o1_ref):
        t = p0_ref[0] + p1_ref[0] + y_ref[...]
        h = jnp.maximum(t * dis_ref[...] + b_ref[...], 0.0)
        o = jnp.dot(h, w_ref[...],
                    preferred_element_type=jnp.float32) * dis_ref[...]
        o_ref[...] = o
        o1_ref[...] = o

    return pl.pallas_call(
        body,
        grid=(g,),
        in_specs=[
            pl.BlockSpec((1, N_BLK, d), lambda i: (0, i, 0)),
            pl.BlockSpec((1, N_BLK, d), lambda i: (1, i, 0)),
            pl.BlockSpec((N_BLK, d), lambda i: (i, 0)),
            pl.BlockSpec((N_BLK, 1), lambda i: (i, 0)),
            pl.BlockSpec((1, d), lambda i: (0, 0)),
            pl.BlockSpec((d, d), lambda i: (0, 0)),
        ],
        out_specs=[
            pl.BlockSpec((N_BLK, d), lambda i: (i, 0)),
            pl.BlockSpec((N_BLK, d), lambda i: (i, 0)),
        ],
        out_shape=[
            jax.ShapeDtypeStruct((n, d), jnp.float32),
            jax.ShapeDtypeStruct((n, d), jnp.float32),
        ],
    )(p, p, y, dis, b.reshape(1, d), w_next)


def _final_tc(p, y, dis, b):
    n, d = y.shape
    g = pl.cdiv(n, N_BLK)

    def body(p0_ref, p1_ref, y_ref, dis_ref, b_ref, o_ref):
        t = p0_ref[0] + p1_ref[0] + y_ref[...]
        o_ref[...] = jnp.maximum(t * dis_ref[...] + b_ref[...], 0.0)

    return pl.pallas_call(
        body,
        grid=(g,),
        in_specs=[
            pl.BlockSpec((1, N_BLK, d), lambda i: (0, i, 0)),
            pl.BlockSpec((1, N_BLK, d), lambda i: (1, i, 0)),
            pl.BlockSpec((N_BLK, d), lambda i: (i, 0)),
            pl.BlockSpec((N_BLK, 1), lambda i: (i, 0)),
            pl.BlockSpec((1, d), lambda i: (0, 0)),
        ],
        out_specs=pl.BlockSpec((N_BLK, d), lambda i: (i, 0)),
        out_shape=jax.ShapeDtypeStruct((n, d), jnp.float32),
    )(p, p, y, dis, b.reshape(1, d))


def kernel(x, edge_index, W1, b1, W2, b2, W3, b3):
    n, d = x.shape
    e = edge_index.shape[1]
    ei = edge_index.astype(jnp.int32)
    per_pair = (-(-e // CHUNK) + NS - 1) // NS
    if per_pair % 2:
        per_pair += 1
    n0 = int(round(per_pair * 0.715))
    n1 = per_pair - n0
    tot = NS * (n0 + n1)
    e_pad = tot * CHUNK
    pad = e_pad - e
    dummy = n + (jnp.arange(pad, dtype=jnp.int32) % 128)
    src_p = jnp.concatenate([ei[0], jnp.zeros((pad,), jnp.int32)])
    dst_p = jnp.concatenate([ei[1], dummy])
    src2 = src_p.reshape(tot, CHUNK)
    dst2 = dst_p.reshape(tot, CHUNK)
    idx4 = jnp.stack([src2, src2 + n, dst2, dst2], axis=1)

    n_hist = n + 144
    hist = _sc_hist(idx4, n_hist)
    y, yb, dis = _prep_tc(hist[:, :n], x, W1)

    n_acc = -(-n // (NS * 8)) * NS * 8 + NS * 8
    p = _sc_aggregate(jnp.concatenate([y, yb], axis=0), idx4, n0, n1, n_acc)
    y, yb = _mid_tc(p, y, dis, b1, W2)
    p = _sc_aggregate(jnp.concatenate([y, yb], axis=0), idx4, n0, n1, n_acc)
    y, yb = _mid_tc(p, y, dis, b2, W3)
    p = _sc_aggregate(jnp.concatenate([y, yb], axis=0), idx4, n0, n1, n_acc)
    return _final_tc(p, y, dis, b3)


# speedup vs baseline: 1.1150x; 1.1150x over previous
import functools

import jax
import jax.numpy as jnp
from jax import lax
from jax.experimental import pallas as pl
from jax.experimental.pallas import tpu as pltpu
from jax.experimental.pallas import tpu_sc as plsc

NC = 2
NS = 16
NW = NC * NS
CHUNK = 128
NBUF = 2
NISLOT = 4
N_BLK = 1024


def _hist_body(idx_hbm, out_hbm, idx_v, hist_v):
    cid = lax.axis_index("c")
    sid = lax.axis_index("s")
    wid = sid * NC + cid
    hc = idx_v.shape[0]
    pltpu.sync_copy(idx_hbm.at[pl.ds(wid * hc, hc)], idx_v)
    nvec = hist_v.shape[0] // 16

    def zero_body(i, c):
        hist_v[pl.ds(i * 16, 16)] = jnp.zeros((16,), jnp.float32)
        return c

    lax.fori_loop(0, nvec, zero_body, 0)
    ones = jnp.ones((16,), jnp.float32)
    nchunk = idx_v.shape[0]

    def chunk_body(j, c):
        for k in range(CHUNK // 16):
            idx = idx_v[j, 2, pl.ds(k * 16, 16)]
            plsc.addupdate_scatter(hist_v, [idx], ones)
        return c

    lax.fori_loop(0, nchunk, chunk_body, 0)
    pltpu.sync_copy(hist_v, out_hbm.at[wid])


def _sc_hist(idx4, n_hist):
    hc = idx4.shape[0] // NW
    mesh = plsc.VectorSubcoreMesh(core_axis_name="c", subcore_axis_name="s")
    f = pl.kernel(
        _hist_body,
        out_type=jax.ShapeDtypeStruct((NW, n_hist), jnp.float32),
        mesh=mesh,
        compiler_params=pltpu.CompilerParams(needs_layout_passes=False),
        scratch_types=[
            pltpu.VMEM((hc, 4, CHUNK), jnp.int32),
            pltpu.VMEM((n_hist,), jnp.float32),
        ],
    )
    return f(idx4)


def _agg_body(n0, n1, y_hbm, idx_hbm, out_hbm,
              ib, rows_v, zsrc, acc_sh, isem, gsem, ssem):
    cid = lax.axis_index("c")
    sid = lax.axis_index("s")
    n_acc = acc_sh.shape[0]
    d = acc_sh.shape[1]
    nis = ib.shape[0]
    nchunk = jnp.where(cid == 0, n0, n1)
    base = jnp.where(cid == 0, sid * n0, NS * n0 + sid * n1)

    pltpu.async_copy(idx_hbm.at[base], ib.at[0], isem.at[0])
    pltpu.async_copy(idx_hbm.at[base + 1], ib.at[1], isem.at[1])

    zr = zsrc.shape[0]

    def zstore(r, c):
        for k in range(CHUNK // 16):
            zsrc[r, pl.ds(k * 16, 16)] = jnp.zeros((16,), jnp.float32)
        return c

    lax.fori_loop(0, zr, zstore, 0)
    zrow = n_acc // NS
    for t in range(zrow // zr):
        pltpu.sync_copy(zsrc, acc_sh.at[pl.ds(sid * zrow + t * zr, zr)])
    plsc.subcore_barrier()

    pltpu.make_async_copy(idx_hbm.at[base], ib.at[0], isem.at[0]).wait()
    pltpu.async_copy(y_hbm.at[ib.at[0, 0]], rows_v.at[0], gsem.at[0])

    def chunk_body(j, c):
        @pl.when(j >= 1)
        def _():
            b = lax.rem(j - 1, NBUF)
            i = lax.rem(j - 1, nis)
            pltpu.make_async_copy(rows_v.at[b], acc_sh.at[ib.at[i, 2]],
                                  ssem.at[b]).wait()

        @pl.when(j + 2 < nchunk)
        def _():
            i = lax.rem(j + 2, nis)
            pltpu.async_copy(idx_hbm.at[base + j + 2], ib.at[i], isem.at[i])

        @pl.when(j + 1 < nchunk)
        def _():
            b = lax.rem(j + 1, NBUF)
            i = lax.rem(j + 1, nis)
            pltpu.make_async_copy(idx_hbm.at[base + j + 1], ib.at[i],
                                  isem.at[i]).wait()
            pltpu.async_copy(y_hbm.at[ib.at[i, 0]], rows_v.at[b], gsem.at[b])

        b = lax.rem(j, NBUF)
        i = lax.rem(j, nis)
        pltpu.make_async_copy(y_hbm.at[ib.at[i, 0]], rows_v.at[b],
                              gsem.at[b]).wait()
        pltpu.async_copy(rows_v.at[b], acc_sh.at[ib.at[i, 2]], ssem.at[b],
                         add=True)
        return c

    lax.fori_loop(0, nchunk, chunk_body, 0)
    j = nchunk - 1
    pltpu.make_async_copy(rows_v.at[lax.rem(j, NBUF)],
                          acc_sh.at[ib.at[lax.rem(j, nis), 2]],
                          ssem.at[lax.rem(j, NBUF)]).wait()
    plsc.subcore_barrier()
    orow = n_acc // NS
    pltpu.sync_copy(acc_sh.at[pl.ds(sid * orow, orow)],
                    out_hbm.at[cid, pl.ds(sid * orow, orow)])


def _sc_aggregate(y, idx4, n0, n1, n_acc):
    d = y.shape[1]
    mesh = plsc.VectorSubcoreMesh(core_axis_name="c", subcore_axis_name="s")
    f = pl.kernel(
        functools.partial(_agg_body, n0, n1),
        out_type=jax.ShapeDtypeStruct((NC, n_acc, d), jnp.float32),
        mesh=mesh,
        scratch_types=[
            pltpu.VMEM((NISLOT, 4, CHUNK), jnp.int32),
            pltpu.VMEM((NBUF, CHUNK, d), jnp.float32),
            pltpu.VMEM((64, d), jnp.float32),
            pltpu.VMEM_SHARED((n_acc, d), jnp.float32),
            pltpu.SemaphoreType.DMA((NISLOT,)),
            pltpu.SemaphoreType.DMA((NBUF,)),
            pltpu.SemaphoreType.DMA((NBUF,)),
        ],
    )
    return f(y, idx4)


def _prep_tc(hist, x, w1):
    n, d = x.shape
    g = pl.cdiv(n, N_BLK)

    def body(hist_ref, x_ref, w_ref, y_ref, dis_ref):
        deg = jnp.sum(hist_ref[...], axis=0) + 1.0
        dis = lax.rsqrt(deg)
        y_ref[...] = jnp.dot(x_ref[...], w_ref[...],
                             preferred_element_type=jnp.float32) * dis[:, None]
        dis_ref[...] = dis[:, None]

    return pl.pallas_call(
        body,
        grid=(g,),
        in_specs=[
            pl.BlockSpec((NW, N_BLK), lambda i: (0, i)),
            pl.BlockSpec((N_BLK, d), lambda i: (i, 0)),
            pl.BlockSpec((d, d), lambda i: (0, 0)),
        ],
        out_specs=[
            pl.BlockSpec((N_BLK, d), lambda i: (i, 0)),
            pl.BlockSpec((N_BLK, 1), lambda i: (i, 0)),
        ],
        out_shape=[
            jax.ShapeDtypeStruct((n, d), jnp.float32),
            jax.ShapeDtypeStruct((n, 1), jnp.float32),
        ],
    )(hist, x, w1)


def _mid_tc(p, y, dis, b, w_next):
    n, d = y.shape
    g = pl.cdiv(n, N_BLK)

    def body(p0_ref, p1_ref, y_ref, dis_ref, b_ref, w_ref, o_ref):
        t = p0_ref[0] + p1_ref[0] + y_ref[...]
        h = jnp.maximum(t * dis_ref[...] + b_ref[...], 0.0)
        o_ref[...] = jnp.dot(h, w_ref[...],
                             preferred_element_type=jnp.float32) * dis_ref[...]

    return pl.pallas_call(
        body,
        grid=(g,),
        in_specs=[
            pl.BlockSpec((1, N_BLK, d), lambda i: (0, i, 0)),
            pl.BlockSpec((1, N_BLK, d), lambda i: (1, i, 0)),
            pl.BlockSpec((N_BLK, d), lambda i: (i, 0)),
            pl.BlockSpec((N_BLK, 1), lambda i: (i, 0)),
            pl.BlockSpec((1, d), lambda i: (0, 0)),
            pl.BlockSpec((d, d), lambda i: (0, 0)),
        ],
        out_specs=pl.BlockSpec((N_BLK, d), lambda i: (i, 0)),
        out_shape=jax.ShapeDtypeStruct((n, d), jnp.float32),
    )(p, p, y, dis, b.reshape(1, d), w_next)


def _final_tc(p, y, dis, b):
    n, d = y.shape
    g = pl.cdiv(n, N_BLK)

    def body(p0_ref, p1_ref, y_ref, dis_ref, b_ref, o_ref):
        t = p0_ref[0] + p1_ref[0] + y_ref[...]
        o_ref[...] = jnp.maximum(t * dis_ref[...] + b_ref[...], 0.0)

    return pl.pallas_call(
        body,
        grid=(g,),
        in_specs=[
            pl.BlockSpec((1, N_BLK, d), lambda i: (0, i, 0)),
            pl.BlockSpec((1, N_BLK, d), lambda i: (1, i, 0)),
            pl.BlockSpec((N_BLK, d), lambda i: (i, 0)),
            pl.BlockSpec((N_BLK, 1), lambda i: (i, 0)),
            pl.BlockSpec((1, d), lambda i: (0, 0)),
        ],
        out_specs=pl.BlockSpec((N_BLK, d), lambda i: (i, 0)),
        out_shape=jax.ShapeDtypeStruct((n, d), jnp.float32),
    )(p, p, y, dis, b.reshape(1, d))


def kernel(x, edge_index, W1, b1, W2, b2, W3, b3):
    n, d = x.shape
    e = edge_index.shape[1]
    ei = edge_index.astype(jnp.int32)
    per_pair = (-(-e // CHUNK) + NS - 1) // NS
    if per_pair % 2:
        per_pair += 1
    n0 = int(round(per_pair * 0.785))
    n1 = per_pair - n0
    tot = NS * (n0 + n1)
    e_pad = tot * CHUNK
    pad = e_pad - e
    dummy = n + (jnp.arange(pad, dtype=jnp.int32) % 128)
    src_p = jnp.concatenate([ei[0], jnp.zeros((pad,), jnp.int32)])
    dst_p = jnp.concatenate([ei[1], dummy])
    src2 = src_p.reshape(tot, CHUNK)
    dst2 = dst_p.reshape(tot, CHUNK)
    idx4 = jnp.stack([src2, src2 + n, dst2, dst2], axis=1)

    n_hist = n + 144
    hist = _sc_hist(idx4, n_hist)
    y, dis = _prep_tc(hist[:, :n], x, W1)

    n_acc = -(-n // (NS * 8)) * NS * 8 + NS * 8
    p = _sc_aggregate(y, idx4, n0, n1, n_acc)
    y = _mid_tc(p, y, dis, b1, W2)
    p = _sc_aggregate(y, idx4, n0, n1, n_acc)
    y = _mid_tc(p, y, dis, b2, W3)
    p = _sc_aggregate(y, idx4, n0, n1, n_acc)
    return _final_tc(p, y, dis, b3)


# speedup vs baseline: 1.3581x; 1.2180x over previous
import functools

import jax
import numpy as np
import jax.numpy as jnp
from jax import lax
from jax.experimental import pallas as pl
from jax.experimental.pallas import tpu as pltpu
from jax.experimental.pallas import tpu_sc as plsc

NC = 2
NS = 16
NW = NC * NS
CHUNK = 112
NBUF = 3
NISLOT = 4
N_BLK = 1024


def _hist_body(idx_hbm, out_hbm, idx_v, hist_v):
    cid = lax.axis_index("c")
    sid = lax.axis_index("s")
    wid = sid * NC + cid
    hc = idx_v.shape[0]
    pltpu.sync_copy(idx_hbm.at[pl.ds(wid * hc, hc)], idx_v)
    nvec = hist_v.shape[0] // 16

    def zero_body(i, c):
        hist_v[pl.ds(i * 16, 16)] = jnp.zeros((16,), jnp.float32)
        return c

    lax.fori_loop(0, nvec, zero_body, 0)
    ones = jnp.ones((16,), jnp.float32)
    nchunk = idx_v.shape[0]

    def chunk_body(j, c):
        for k in range(CHUNK // 16):
            idx = idx_v[j, 1, pl.ds(k * 16, 16)]
            plsc.addupdate_scatter(hist_v, [idx], ones)
        return c

    lax.fori_loop(0, nchunk, chunk_body, 0)
    pltpu.sync_copy(hist_v, out_hbm.at[wid])


def _sc_hist(idx4, n_hist):
    hc = idx4.shape[0] // NW
    mesh = plsc.VectorSubcoreMesh(core_axis_name="c", subcore_axis_name="s")
    f = pl.kernel(
        _hist_body,
        out_type=jax.ShapeDtypeStruct((NW, n_hist), jnp.float32),
        mesh=mesh,
        compiler_params=pltpu.CompilerParams(needs_layout_passes=False),
        scratch_types=[
            pltpu.VMEM((hc, 2, CHUNK), jnp.int32),
            pltpu.VMEM((n_hist,), jnp.float32),
        ],
    )
    return f(idx4)


def _agg_body(n0, n1, y_hbm, idx_hbm, out_hbm,
              ib, fr, acc_sh, isem, gsem, ssem):
    cid = lax.axis_index("c")
    sid = lax.axis_index("s")
    n_acc = acc_sh.shape[0]
    d = acc_sh.shape[1]
    nis = ib.shape[0]
    nchunk = jnp.where(cid == 0, n0, n1)
    base = jnp.where(cid == 0, sid * n0, NS * n0 + sid * n1)

    for i in range(3):
        pltpu.async_copy(idx_hbm.at[base + i], ib.at[i], isem.at[i])

    def zstore(r, c):
        for k in range(d // 16):
            fr[0, r, pl.ds(k * 16, 16)] = jnp.zeros((16,), jnp.float32)
        return c

    lax.fori_loop(0, CHUNK, zstore, 0)
    zrow = n_acc // NS
    nfull = zrow // CHUNK
    rem = zrow - nfull * CHUNK
    for t in range(nfull):
        pltpu.sync_copy(fr.at[0],
                        acc_sh.at[pl.ds(sid * zrow + t * CHUNK, CHUNK)])
    if rem:
        pltpu.sync_copy(fr.at[0, pl.ds(0, rem)],
                        acc_sh.at[pl.ds(sid * zrow + nfull * CHUNK, rem)])
    plsc.subcore_barrier()

    for i in range(2):
        pltpu.make_async_copy(idx_hbm.at[base + i], ib.at[i], isem.at[i]).wait()
        pltpu.async_copy(y_hbm.at[ib.at[i, 0]], fr.at[i], gsem.at[i])

    def chunk_body(j, c):
        @pl.when(j >= 1)
        def _():
            b = lax.rem(j - 1, NBUF)
            i = lax.rem(j - 1, nis)
            pltpu.make_async_copy(fr.at[b], acc_sh.at[ib.at[i, 1]],
                                  ssem.at[b]).wait()

        @pl.when(j + 3 < nchunk)
        def _():
            i = lax.rem(j + 3, nis)
            pltpu.async_copy(idx_hbm.at[base + j + 3], ib.at[i], isem.at[i])

        @pl.when(j + 2 < nchunk)
        def _():
            b = lax.rem(j + 2, NBUF)
            i = lax.rem(j + 2, nis)
            pltpu.make_async_copy(idx_hbm.at[base + j + 2], ib.at[i],
                                  isem.at[i]).wait()
            pltpu.async_copy(y_hbm.at[ib.at[i, 0]], fr.at[b], gsem.at[b])

        b = lax.rem(j, NBUF)
        i = lax.rem(j, nis)
        pltpu.make_async_copy(y_hbm.at[ib.at[i, 0]], fr.at[b],
                              gsem.at[b]).wait()
        pltpu.async_copy(fr.at[b], acc_sh.at[ib.at[i, 1]], ssem.at[b],
                         add=True)
        return c

    lax.fori_loop(0, nchunk, chunk_body, 0)
    j = nchunk - 1
    pltpu.make_async_copy(fr.at[lax.rem(j, NBUF)],
                          acc_sh.at[ib.at[lax.rem(j, nis), 1]],
                          ssem.at[lax.rem(j, NBUF)]).wait()
    plsc.subcore_barrier()
    orow = n_acc // NS
    pltpu.sync_copy(acc_sh.at[pl.ds(sid * orow, orow)],
                    out_hbm.at[cid, pl.ds(sid * orow, orow)])


def _sc_aggregate(y, idx4, n0, n1, n_acc):
    d = y.shape[1]
    mesh = plsc.VectorSubcoreMesh(core_axis_name="c", subcore_axis_name="s")
    f = pl.kernel(
        functools.partial(_agg_body, n0, n1),
        out_type=jax.ShapeDtypeStruct((NC, n_acc, d), jnp.float32),
        mesh=mesh,
        scratch_types=[
            pltpu.VMEM((NISLOT, 2, CHUNK), jnp.int32),
            pltpu.VMEM((NBUF, CHUNK, d), jnp.float32),
            pltpu.VMEM_SHARED((n_acc, d), jnp.float32),
            pltpu.SemaphoreType.DMA((NISLOT,)),
            pltpu.SemaphoreType.DMA((NBUF,)),
            pltpu.SemaphoreType.DMA((NBUF,)),
        ],
    )
    return f(y, idx4)


def _prep_tc(hist, x, w1):
    n, d = x.shape
    g = pl.cdiv(n, N_BLK)

    def body(hist_ref, x_ref, w_ref, y_ref, dis_ref):
        deg = jnp.sum(hist_ref[...], axis=0) + 1.0
        dis = lax.rsqrt(deg)
        y_ref[...] = jnp.dot(x_ref[...], w_ref[...],
                             preferred_element_type=jnp.float32) * dis[:, None]
        dis_ref[...] = dis[:, None]

    return pl.pallas_call(
        body,
        grid=(g,),
        in_specs=[
            pl.BlockSpec((NW, N_BLK), lambda i: (0, i)),
            pl.BlockSpec((N_BLK, d), lambda i: (i, 0)),
            pl.BlockSpec((d, d), lambda i: (0, 0)),
        ],
        out_specs=[
            pl.BlockSpec((N_BLK, d), lambda i: (i, 0)),
            pl.BlockSpec((N_BLK, 1), lambda i: (i, 0)),
        ],
        out_shape=[
            jax.ShapeDtypeStruct((n, d), jnp.float32),
            jax.ShapeDtypeStruct((n, 1), jnp.float32),
        ],
    )(hist, x, w1)


def _mid_tc(p, y, dis, b, w_next):
    n, d = y.shape
    g = pl.cdiv(n, N_BLK)

    def body(p0_ref, p1_ref, y_ref, dis_ref, b_ref, w_ref, o_ref):
        t = p0_ref[0] + p1_ref[0] + y_ref[...]
        h = jnp.maximum(t * dis_ref[...] + b_ref[...], 0.0)
        o_ref[...] = jnp.dot(h, w_ref[...],
                             preferred_element_type=jnp.float32) * dis_ref[...]

    return pl.pallas_call(
        body,
        grid=(g,),
        in_specs=[
            pl.BlockSpec((1, N_BLK, d), lambda i: (0, i, 0)),
            pl.BlockSpec((1, N_BLK, d), lambda i: (1, i, 0)),
            pl.BlockSpec((N_BLK, d), lambda i: (i, 0)),
            pl.BlockSpec((N_BLK, 1), lambda i: (i, 0)),
            pl.BlockSpec((1, d), lambda i: (0, 0)),
            pl.BlockSpec((d, d), lambda i: (0, 0)),
        ],
        out_specs=pl.BlockSpec((N_BLK, d), lambda i: (i, 0)),
        out_shape=jax.ShapeDtypeStruct((n, d), jnp.float32),
    )(p, p, y, dis, b.reshape(1, d), w_next)


def _final_tc(p, y, dis, b):
    n, d = y.shape
    g = pl.cdiv(n, N_BLK)

    def body(p0_ref, p1_ref, y_ref, dis_ref, b_ref, o_ref):
        t = p0_ref[0] + p1_ref[0] + y_ref[...]
        o_ref[...] = jnp.maximum(t * dis_ref[...] + b_ref[...], 0.0)

    return pl.pallas_call(
        body,
        grid=(g,),
        in_specs=[
            pl.BlockSpec((1, N_BLK, d), lambda i: (0, i, 0)),
            pl.BlockSpec((1, N_BLK, d), lambda i: (1, i, 0)),
            pl.BlockSpec((N_BLK, d), lambda i: (i, 0)),
            pl.BlockSpec((N_BLK, 1), lambda i: (i, 0)),
            pl.BlockSpec((1, d), lambda i: (0, 0)),
        ],
        out_specs=pl.BlockSpec((N_BLK, d), lambda i: (i, 0)),
        out_shape=jax.ShapeDtypeStruct((n, d), jnp.float32),
    )(p, p, y, dis, b.reshape(1, d))


def kernel(x, edge_index, W1, b1, W2, b2, W3, b3):
    n, d = x.shape
    e = edge_index.shape[1]
    ei = edge_index.astype(jnp.int32)
    per_pair = (-(-e // CHUNK) + NS - 1) // NS
    if per_pair % 2:
        per_pair += 1
    n0 = int(round(per_pair * 0.785))
    n1 = per_pair - n0
    tot = NS * (n0 + n1)
    e_pad = tot * CHUNK
    pad = e_pad - e
    dummy = n + (jnp.arange(pad, dtype=jnp.int32) % 112)
    src_p = jnp.concatenate([ei[0], jnp.zeros((pad,), jnp.int32)])
    dst_p = jnp.concatenate([ei[1], dummy])
    idx4 = jnp.stack([src_p.reshape(tot, CHUNK),
                      dst_p.reshape(tot, CHUNK)], axis=1)

    n_hist = n + 144
    hist = _sc_hist(idx4, n_hist)
    y, dis = _prep_tc(hist[:, :n], x, W1)

    n_acc = -(-n // (NS * 8)) * NS * 8
    p = _sc_aggregate(y, idx4, n0, n1, n_acc)
    y = _mid_tc(p, y, dis, b1, W2)
    p = _sc_aggregate(y, idx4, n0, n1, n_acc)
    y = _mid_tc(p, y, dis, b2, W3)
    p = _sc_aggregate(y, idx4, n0, n1, n_acc)
    return _final_tc(p, y, dis, b3)


# speedup vs baseline: 1.3779x; 1.0146x over previous
import functools

import jax
import numpy as np
import jax.numpy as jnp
from jax import lax
from jax.experimental import pallas as pl
from jax.experimental.pallas import tpu as pltpu
from jax.experimental.pallas import tpu_sc as plsc

NC = 2
NS = 16
NW = NC * NS
CHUNK = 112
NBUF = 3
NISLOT = 4
N_BLK = 1024


def _hist_body(idx_hbm, out_hbm, idx_v, hist_v):
    cid = lax.axis_index("c")
    sid = lax.axis_index("s")
    wid = sid * NC + cid
    hc = idx_v.shape[0]
    pltpu.sync_copy(idx_hbm.at[pl.ds(wid * hc, hc)], idx_v)
    nvec = hist_v.shape[0] // 16

    def zero_body(i, c):
        hist_v[pl.ds(i * 16, 16)] = jnp.zeros((16,), jnp.float32)
        return c

    lax.fori_loop(0, nvec, zero_body, 0)
    ones = jnp.ones((16,), jnp.float32)
    nchunk = idx_v.shape[0]

    def chunk_body(j, c):
        for k in range(CHUNK // 16):
            idx = idx_v[j, 1, pl.ds(k * 16, 16)]
            plsc.addupdate_scatter(hist_v, [idx], ones)
        return c

    lax.fori_loop(0, nchunk, chunk_body, 0)
    pltpu.sync_copy(hist_v, out_hbm.at[wid])


def _sc_hist(idx4, n_hist):
    hc = idx4.shape[0] // NW
    mesh = plsc.VectorSubcoreMesh(core_axis_name="c", subcore_axis_name="s")
    f = pl.kernel(
        _hist_body,
        out_type=jax.ShapeDtypeStruct((NW, n_hist), jnp.float32),
        mesh=mesh,
        compiler_params=pltpu.CompilerParams(needs_layout_passes=False),
        scratch_types=[
            pltpu.VMEM((hc, 2, CHUNK), jnp.int32),
            pltpu.VMEM((n_hist,), jnp.float32),
        ],
    )
    return f(idx4)


def _agg_body(n0, n1, y_hbm, idx_hbm, out_hbm,
              ib, fr, acc_sh, isem, gsem, ssem):
    cid = lax.axis_index("c")
    sid = lax.axis_index("s")
    n_acc = acc_sh.shape[0]
    d = acc_sh.shape[1]
    nis = ib.shape[0]
    nchunk = jnp.where(cid == 0, n0, n1)
    base = jnp.where(cid == 0, sid * n0, NS * n0 + sid * n1)

    for i in range(3):
        pltpu.async_copy(idx_hbm.at[base + i], ib.at[i], isem.at[i])

    def zstore(r, c):
        for k in range(d // 16):
            fr[0, r, pl.ds(k * 16, 16)] = jnp.zeros((16,), jnp.float32)
        return c

    lax.fori_loop(0, CHUNK, zstore, 0)
    zrow = n_acc // NS
    nfull = zrow // CHUNK
    rem = zrow - nfull * CHUNK
    for t in range(nfull):
        pltpu.sync_copy(fr.at[0],
                        acc_sh.at[pl.ds(sid * zrow + t * CHUNK, CHUNK)])
    if rem:
        pltpu.sync_copy(fr.at[0, pl.ds(0, rem)],
                        acc_sh.at[pl.ds(sid * zrow + nfull * CHUNK, rem)])
    plsc.subcore_barrier()

    for i in range(2):
        pltpu.make_async_copy(idx_hbm.at[base + i], ib.at[i], isem.at[i]).wait()
        pltpu.async_copy(y_hbm.at[ib.at[i, 0]], fr.at[i], gsem.at[i])

    def chunk_body(j, c):
        @pl.when(j >= 1)
        def _():
            b = lax.rem(j - 1, NBUF)
            i = lax.rem(j - 1, nis)
            pltpu.make_async_copy(fr.at[b], acc_sh.at[ib.at[i, 1]],
                                  ssem.at[b]).wait()

        @pl.when(j + 3 < nchunk)
        def _():
            i = lax.rem(j + 3, nis)
            pltpu.async_copy(idx_hbm.at[base + j + 3], ib.at[i], isem.at[i])

        @pl.when(j + 2 < nchunk)
        def _():
            b = lax.rem(j + 2, NBUF)
            i = lax.rem(j + 2, nis)
            pltpu.make_async_copy(idx_hbm.at[base + j + 2], ib.at[i],
                                  isem.at[i]).wait()
            pltpu.async_copy(y_hbm.at[ib.at[i, 0]], fr.at[b], gsem.at[b])

        b = lax.rem(j, NBUF)
        i = lax.rem(j, nis)
        pltpu.make_async_copy(y_hbm.at[ib.at[i, 0]], fr.at[b],
                              gsem.at[b]).wait()
        pltpu.async_copy(fr.at[b], acc_sh.at[ib.at[i, 1]], ssem.at[b],
                         add=True)
        return c

    lax.fori_loop(0, nchunk, chunk_body, 0)
    j = nchunk - 1
    pltpu.make_async_copy(fr.at[lax.rem(j, NBUF)],
                          acc_sh.at[ib.at[lax.rem(j, nis), 1]],
                          ssem.at[lax.rem(j, NBUF)]).wait()
    plsc.subcore_barrier()
    orow = n_acc // NS
    pltpu.sync_copy(acc_sh.at[pl.ds(sid * orow, orow)],
                    out_hbm.at[cid, pl.ds(sid * orow, orow)])


def _sc_aggregate(y, idx4, n0, n1, n_acc):
    d = y.shape[1]
    mesh = plsc.VectorSubcoreMesh(core_axis_name="c", subcore_axis_name="s")
    f = pl.kernel(
        functools.partial(_agg_body, n0, n1),
        out_type=jax.ShapeDtypeStruct((NC, n_acc, d), jnp.float32),
        mesh=mesh,
        scratch_types=[
            pltpu.VMEM((NISLOT, 2, CHUNK), jnp.int32),
            pltpu.VMEM((NBUF, CHUNK, d), jnp.float32),
            pltpu.VMEM_SHARED((n_acc, d), jnp.float32),
            pltpu.SemaphoreType.DMA((NISLOT,)),
            pltpu.SemaphoreType.DMA((NBUF,)),
            pltpu.SemaphoreType.DMA((NBUF,)),
        ],
    )
    return f(y, idx4)


def _prep_tc(hist, x, w1):
    n, d = x.shape
    g = pl.cdiv(n, N_BLK)

    def body(hist_ref, x_ref, w_ref, y_ref, dis_ref):
        deg = jnp.sum(hist_ref[...], axis=0) + 1.0
        dis = lax.rsqrt(deg)
        y_ref[...] = jnp.dot(x_ref[...], w_ref[...],
                             preferred_element_type=jnp.float32) * dis[:, None]
        dis_ref[...] = dis[:, None]

    return pl.pallas_call(
        body,
        grid=(g,),
        in_specs=[
            pl.BlockSpec((NW, N_BLK), lambda i: (0, i)),
            pl.BlockSpec((N_BLK, d), lambda i: (i, 0)),
            pl.BlockSpec((d, d), lambda i: (0, 0)),
        ],
        out_specs=[
            pl.BlockSpec((N_BLK, d), lambda i: (i, 0)),
            pl.BlockSpec((N_BLK, 1), lambda i: (i, 0)),
        ],
        out_shape=[
            jax.ShapeDtypeStruct((n, d), jnp.float32),
            jax.ShapeDtypeStruct((n, 1), jnp.float32),
        ],
    )(hist, x, w1)


def _mid_tc(p, y, dis, b, w_next):
    n, d = y.shape
    g = pl.cdiv(n, N_BLK)

    def body(p0_ref, p1_ref, y_ref, dis_ref, b_ref, w_ref, o_ref):
        t = p0_ref[0] + p1_ref[0] + y_ref[...]
        h = jnp.maximum(t * dis_ref[...] + b_ref[...], 0.0)
        o_ref[...] = jnp.dot(h, w_ref[...],
                             preferred_element_type=jnp.float32) * dis_ref[...]

    return pl.pallas_call(
        body,
        grid=(g,),
        in_specs=[
            pl.BlockSpec((1, N_BLK, d), lambda i: (0, i, 0)),
            pl.BlockSpec((1, N_BLK, d), lambda i: (1, i, 0)),
            pl.BlockSpec((N_BLK, d), lambda i: (i, 0)),
            pl.BlockSpec((N_BLK, 1), lambda i: (i, 0)),
            pl.BlockSpec((1, d), lambda i: (0, 0)),
            pl.BlockSpec((d, d), lambda i: (0, 0)),
        ],
        out_specs=pl.BlockSpec((N_BLK, d), lambda i: (i, 0)),
        out_shape=jax.ShapeDtypeStruct((n, d), jnp.float32),
    )(p, p, y, dis, b.reshape(1, d), w_next)


def _final_tc(p, y, dis, b):
    n, d = y.shape
    g = pl.cdiv(n, N_BLK)

    def body(p0_ref, p1_ref, y_ref, dis_ref, b_ref, o_ref):
        t = p0_ref[0] + p1_ref[0] + y_ref[...]
        o_ref[...] = jnp.maximum(t * dis_ref[...] + b_ref[...], 0.0)

    return pl.pallas_call(
        body,
        grid=(g,),
        in_specs=[
            pl.BlockSpec((1, N_BLK, d), lambda i: (0, i, 0)),
            pl.BlockSpec((1, N_BLK, d), lambda i: (1, i, 0)),
            pl.BlockSpec((N_BLK, d), lambda i: (i, 0)),
            pl.BlockSpec((N_BLK, 1), lambda i: (i, 0)),
            pl.BlockSpec((1, d), lambda i: (0, 0)),
        ],
        out_specs=pl.BlockSpec((N_BLK, d), lambda i: (i, 0)),
        out_shape=jax.ShapeDtypeStruct((n, d), jnp.float32),
    )(p, p, y, dis, b.reshape(1, d))


def kernel(x, edge_index, W1, b1, W2, b2, W3, b3):
    n, d = x.shape
    e = edge_index.shape[1]
    ei = edge_index.astype(jnp.int32)
    per_pair = (-(-e // CHUNK) + NS - 1) // NS
    if per_pair % 2:
        per_pair += 1
    n0 = int(round(per_pair * 0.82))
    n1 = per_pair - n0
    tot = NS * (n0 + n1)
    e_pad = tot * CHUNK
    pad = e_pad - e
    dummy = n + (jnp.arange(pad, dtype=jnp.int32) % 112)
    src_p = jnp.concatenate([ei[0], jnp.zeros((pad,), jnp.int32)])
    dst_p = jnp.concatenate([ei[1], dummy])
    idx4 = jnp.stack([src_p.reshape(tot, CHUNK),
                      dst_p.reshape(tot, CHUNK)], axis=1)

    n_hist = n + 144
    hist = _sc_hist(idx4, n_hist)
    y, dis = _prep_tc(hist[:, :n], x, W1)

    n_acc = -(-n // (NS * 8)) * NS * 8
    p = _sc_aggregate(y, idx4, n0, n1, n_acc)
    y = _mid_tc(p, y, dis, b1, W2)
    p = _sc_aggregate(y, idx4, n0, n1, n_acc)
    y = _mid_tc(p, y, dis, b2, W3)
    p = _sc_aggregate(y, idx4, n0, n1, n_acc)
    return _final_tc(p, y, dis, b3)


# speedup vs baseline: 1.4061x; 1.0204x over previous
import functools

import jax
import numpy as np
import jax.numpy as jnp
from jax import lax
from jax.experimental import pallas as pl
from jax.experimental.pallas import tpu as pltpu
from jax.experimental.pallas import tpu_sc as plsc

NC = 2
NS = 16
NW = NC * NS
CHUNK = 112
NBUF = 3
NISLOT = 4
N_BLK = 1024


def _hist_body(idx_hbm, out_hbm, idx_v, hist_v):
    cid = lax.axis_index("c")
    sid = lax.axis_index("s")
    wid = sid * NC + cid
    hc = idx_v.shape[0]
    pltpu.sync_copy(idx_hbm.at[pl.ds(wid * hc, hc)], idx_v)
    nvec = hist_v.shape[0] // 16

    def zero_body(i, c):
        hist_v[pl.ds(i * 16, 16)] = jnp.zeros((16,), jnp.float32)
        return c

    lax.fori_loop(0, nvec, zero_body, 0)
    ones = jnp.ones((16,), jnp.float32)
    nchunk = idx_v.shape[0]

    def chunk_body(j, c):
        for k in range(CHUNK // 16):
            idx = idx_v[j, 1, pl.ds(k * 16, 16)]
            plsc.addupdate_scatter(hist_v, [idx], ones)
        return c

    lax.fori_loop(0, nchunk, chunk_body, 0)
    pltpu.sync_copy(hist_v, out_hbm.at[wid])


def _sc_hist(idx4, n_hist):
    hc = idx4.shape[0] // NW
    mesh = plsc.VectorSubcoreMesh(core_axis_name="c", subcore_axis_name="s")
    f = pl.kernel(
        _hist_body,
        out_type=jax.ShapeDtypeStruct((NW, n_hist), jnp.float32),
        mesh=mesh,
        compiler_params=pltpu.CompilerParams(needs_layout_passes=False),
        scratch_types=[
            pltpu.VMEM((hc, 2, CHUNK), jnp.int32),
            pltpu.VMEM((n_hist,), jnp.float32),
        ],
    )
    return f(idx4)


def _agg_body(n0, n1, y_hbm, idx_hbm, out_hbm,
              ib, fr, acc_sh, isem, gsem, ssem):
    cid = lax.axis_index("c")
    sid = lax.axis_index("s")
    n_acc = acc_sh.shape[0]
    d = acc_sh.shape[1]
    nis = ib.shape[0]
    nchunk = jnp.where(cid == 0, n0, n1)
    base = jnp.where(cid == 0, sid * n0, NS * n0 + sid * n1)

    for i in range(3):
        pltpu.async_copy(idx_hbm.at[base + i], ib.at[i], isem.at[i])

    def zstore(r, c):
        for k in range(d // 16):
            fr[0, r, pl.ds(k * 16, 16)] = jnp.zeros((16,), jnp.float32)
        return c

    lax.fori_loop(0, CHUNK, zstore, 0)
    zrow = n_acc // NS
    nfull = zrow // CHUNK
    rem = zrow - nfull * CHUNK
    for t in range(nfull):
        pltpu.sync_copy(fr.at[0],
                        acc_sh.at[pl.ds(sid * zrow + t * CHUNK, CHUNK)])
    if rem:
        pltpu.sync_copy(fr.at[0, pl.ds(0, rem)],
                        acc_sh.at[pl.ds(sid * zrow + nfull * CHUNK, rem)])
    plsc.subcore_barrier()

    for i in range(2):
        pltpu.make_async_copy(idx_hbm.at[base + i], ib.at[i], isem.at[i]).wait()
        pltpu.async_copy(y_hbm.at[ib.at[i, 0]], fr.at[i], gsem.at[i])

    def chunk_body(j, c):
        @pl.when(j >= 1)
        def _():
            b = lax.rem(j - 1, NBUF)
            i = lax.rem(j - 1, nis)
            pltpu.make_async_copy(fr.at[b], acc_sh.at[ib.at[i, 1]],
                                  ssem.at[b]).wait()

        @pl.when(j + 3 < nchunk)
        def _():
            i = lax.rem(j + 3, nis)
            pltpu.async_copy(idx_hbm.at[base + j + 3], ib.at[i], isem.at[i])

        @pl.when(j + 2 < nchunk)
        def _():
            b = lax.rem(j + 2, NBUF)
            i = lax.rem(j + 2, nis)
            pltpu.make_async_copy(idx_hbm.at[base + j + 2], ib.at[i],
                                  isem.at[i]).wait()
            pltpu.async_copy(y_hbm.at[ib.at[i, 0]], fr.at[b], gsem.at[b])

        b = lax.rem(j, NBUF)
        i = lax.rem(j, nis)
        pltpu.make_async_copy(y_hbm.at[ib.at[i, 0]], fr.at[b],
                              gsem.at[b]).wait()
        pltpu.async_copy(fr.at[b], acc_sh.at[ib.at[i, 1]], ssem.at[b],
                         add=True)
        return c

    lax.fori_loop(0, nchunk, chunk_body, 0)
    j = nchunk - 1
    pltpu.make_async_copy(fr.at[lax.rem(j, NBUF)],
                          acc_sh.at[ib.at[lax.rem(j, nis), 1]],
                          ssem.at[lax.rem(j, NBUF)]).wait()
    plsc.subcore_barrier()
    orow = n_acc // NS
    pltpu.sync_copy(acc_sh.at[pl.ds(sid * orow, orow)],
                    out_hbm.at[cid, pl.ds(sid * orow, orow)])


def _sc_aggregate(y, idx4, n0, n1, n_acc):
    d = y.shape[1]
    mesh = plsc.VectorSubcoreMesh(core_axis_name="c", subcore_axis_name="s")
    f = pl.kernel(
        functools.partial(_agg_body, n0, n1),
        out_type=jax.ShapeDtypeStruct((NC, n_acc, d), jnp.float32),
        mesh=mesh,
        scratch_types=[
            pltpu.VMEM((NISLOT, 2, CHUNK), jnp.int32),
            pltpu.VMEM((NBUF, CHUNK, d), jnp.float32),
            pltpu.VMEM_SHARED((n_acc, d), jnp.float32),
            pltpu.SemaphoreType.DMA((NISLOT,)),
            pltpu.SemaphoreType.DMA((NBUF,)),
            pltpu.SemaphoreType.DMA((NBUF,)),
        ],
    )
    return f(y, idx4)


def _prep_tc(hist, x, w1):
    n, d = x.shape
    g = pl.cdiv(n, N_BLK)

    def body(hist_ref, x_ref, w_ref, y_ref, dis_ref):
        deg = jnp.sum(hist_ref[...], axis=0) + 1.0
        dis = lax.rsqrt(deg)
        y_ref[...] = jnp.dot(x_ref[...], w_ref[...],
                             preferred_element_type=jnp.float32) * dis[:, None]
        dis_ref[...] = dis[:, None]

    return pl.pallas_call(
        body,
        grid=(g,),
        in_specs=[
            pl.BlockSpec((NW, N_BLK), lambda i: (0, i)),
            pl.BlockSpec((N_BLK, d), lambda i: (i, 0)),
            pl.BlockSpec((d, d), lambda i: (0, 0)),
        ],
        out_specs=[
            pl.BlockSpec((N_BLK, d), lambda i: (i, 0)),
            pl.BlockSpec((N_BLK, 1), lambda i: (i, 0)),
        ],
        out_shape=[
            jax.ShapeDtypeStruct((n, d), jnp.float32),
            jax.ShapeDtypeStruct((n, 1), jnp.float32),
        ],
    )(hist, x, w1)


def _mid_tc(p, y, dis, b, w_next):
    n, d = y.shape
    g = pl.cdiv(n, N_BLK)

    def body(p0_ref, p1_ref, y_ref, dis_ref, b_ref, w_ref, o_ref):
        t = p0_ref[0] + p1_ref[0] + y_ref[...]
        h = jnp.maximum(t * dis_ref[...] + b_ref[...], 0.0)
        o_ref[...] = jnp.dot(h, w_ref[...],
                             preferred_element_type=jnp.float32) * dis_ref[...]

    return pl.pallas_call(
        body,
        grid=(g,),
        in_specs=[
            pl.BlockSpec((1, N_BLK, d), lambda i: (0, i, 0)),
            pl.BlockSpec((1, N_BLK, d), lambda i: (1, i, 0)),
            pl.BlockSpec((N_BLK, d), lambda i: (i, 0)),
            pl.BlockSpec((N_BLK, 1), lambda i: (i, 0)),
            pl.BlockSpec((1, d), lambda i: (0, 0)),
            pl.BlockSpec((d, d), lambda i: (0, 0)),
        ],
        out_specs=pl.BlockSpec((N_BLK, d), lambda i: (i, 0)),
        out_shape=jax.ShapeDtypeStruct((n, d), jnp.float32),
    )(p, p, y, dis, b.reshape(1, d), w_next)


def _final_tc(p, y, dis, b):
    n, d = y.shape
    g = pl.cdiv(n, N_BLK)

    def body(p0_ref, p1_ref, y_ref, dis_ref, b_ref, o_ref):
        t = p0_ref[0] + p1_ref[0] + y_ref[...]
        o_ref[...] = jnp.maximum(t * dis_ref[...] + b_ref[...], 0.0)

    return pl.pallas_call(
        body,
        grid=(g,),
        in_specs=[
            pl.BlockSpec((1, N_BLK, d), lambda i: (0, i, 0)),
            pl.BlockSpec((1, N_BLK, d), lambda i: (1, i, 0)),
            pl.BlockSpec((N_BLK, d), lambda i: (i, 0)),
            pl.BlockSpec((N_BLK, 1), lambda i: (i, 0)),
            pl.BlockSpec((1, d), lambda i: (0, 0)),
        ],
        out_specs=pl.BlockSpec((N_BLK, d), lambda i: (i, 0)),
        out_shape=jax.ShapeDtypeStruct((n, d), jnp.float32),
    )(p, p, y, dis, b.reshape(1, d))


def kernel(x, edge_index, W1, b1, W2, b2, W3, b3):
    n, d = x.shape
    e = edge_index.shape[1]
    ei = edge_index.astype(jnp.int32)
    per_pair = (-(-e // CHUNK) + NS - 1) // NS
    if per_pair % 2:
        per_pair += 1
    n0 = int(round(per_pair * 0.86))
    n1 = per_pair - n0
    tot = NS * (n0 + n1)
    e_pad = tot * CHUNK
    pad = e_pad - e
    dummy = n + (jnp.arange(pad, dtype=jnp.int32) % 112)
    src_p = jnp.concatenate([ei[0], jnp.zeros((pad,), jnp.int32)])
    dst_p = jnp.concatenate([ei[1], dummy])
    idx4 = jnp.stack([src_p.reshape(tot, CHUNK),
                      dst_p.reshape(tot, CHUNK)], axis=1)

    n_hist = n + 144
    hist = _sc_hist(idx4, n_hist)
    y, dis = _prep_tc(hist[:, :n], x, W1)

    n_acc = -(-n // (NS * 8)) * NS * 8
    p = _sc_aggregate(y, idx4, n0, n1, n_acc)
    y = _mid_tc(p, y, dis, b1, W2)
    p = _sc_aggregate(y, idx4, n0, n1, n_acc)
    y = _mid_tc(p, y, dis, b2, W3)
    p = _sc_aggregate(y, idx4, n0, n1, n_acc)
    return _final_tc(p, y, dis, b3)


# speedup vs baseline: 1.4112x; 1.0037x over previous
import functools

import jax
import numpy as np
import jax.numpy as jnp
from jax import lax
from jax.experimental import pallas as pl
from jax.experimental.pallas import tpu as pltpu
from jax.experimental.pallas import tpu_sc as plsc

NC = 2
NS = 16
NW = NC * NS
CHUNK = 112
NBUF = 3
NISLOT = 4
N_BLK = 1024


def _hist_body(idx_hbm, out_hbm, idx_v, hist_v):
    cid = lax.axis_index("c")
    sid = lax.axis_index("s")
    wid = sid * NC + cid
    hc = idx_v.shape[0]
    pltpu.sync_copy(idx_hbm.at[pl.ds(wid * hc, hc)], idx_v)
    nvec = hist_v.shape[0] // 16

    def zero_body(i, c):
        hist_v[pl.ds(i * 16, 16)] = jnp.zeros((16,), jnp.float32)
        return c

    lax.fori_loop(0, nvec, zero_body, 0)
    ones = jnp.ones((16,), jnp.float32)
    nchunk = idx_v.shape[0]

    def chunk_body(j, c):
        for k in range(CHUNK // 16):
            idx = idx_v[j, 1, pl.ds(k * 16, 16)]
            plsc.addupdate_scatter(hist_v, [idx], ones)
        return c

    lax.fori_loop(0, nchunk, chunk_body, 0)
    pltpu.sync_copy(hist_v, out_hbm.at[wid])


def _sc_hist(idx4, n_hist):
    hc = idx4.shape[0] // NW
    mesh = plsc.VectorSubcoreMesh(core_axis_name="c", subcore_axis_name="s")
    f = pl.kernel(
        _hist_body,
        out_type=jax.ShapeDtypeStruct((NW, n_hist), jnp.float32),
        mesh=mesh,
        compiler_params=pltpu.CompilerParams(needs_layout_passes=False),
        scratch_types=[
            pltpu.VMEM((hc, 2, CHUNK), jnp.int32),
            pltpu.VMEM((n_hist,), jnp.float32),
        ],
    )
    return f(idx4)


def _agg_body(n0, n1, y_hbm, idx_hbm, out_hbm,
              ib, fr, acc_sh, isem, gsem, ssem):
    cid = lax.axis_index("c")
    sid = lax.axis_index("s")
    n_acc = acc_sh.shape[0]
    d = acc_sh.shape[1]
    nis = ib.shape[0]
    nchunk = jnp.where(cid == 0, n0, n1)
    base = jnp.where(cid == 0, sid * n0, NS * n0 + sid * n1)

    for i in range(3):
        pltpu.async_copy(idx_hbm.at[base + i], ib.at[i], isem.at[i])

    def zstore(r, c):
        for k in range(d // 16):
            fr[0, r, pl.ds(k * 16, 16)] = jnp.zeros((16,), jnp.float32)
        return c

    lax.fori_loop(0, CHUNK, zstore, 0)
    zrow = n_acc // NS
    nfull = zrow // CHUNK
    rem = zrow - nfull * CHUNK
    for t in range(nfull):
        pltpu.sync_copy(fr.at[0],
                        acc_sh.at[pl.ds(sid * zrow + t * CHUNK, CHUNK)])
    if rem:
        pltpu.sync_copy(fr.at[0, pl.ds(0, rem)],
                        acc_sh.at[pl.ds(sid * zrow + nfull * CHUNK, rem)])
    plsc.subcore_barrier()

    for i in range(2):
        pltpu.make_async_copy(idx_hbm.at[base + i], ib.at[i], isem.at[i]).wait()
        pltpu.async_copy(y_hbm.at[ib.at[i, 0]], fr.at[i], gsem.at[i])

    def chunk_body(j, c):
        @pl.when(j >= 1)
        def _():
            b = lax.rem(j - 1, NBUF)
            i = lax.rem(j - 1, nis)
            pltpu.make_async_copy(fr.at[b], acc_sh.at[ib.at[i, 1]],
                                  ssem.at[b]).wait()

        @pl.when(j + 3 < nchunk)
        def _():
            i = lax.rem(j + 3, nis)
            pltpu.async_copy(idx_hbm.at[base + j + 3], ib.at[i], isem.at[i])

        @pl.when(j + 2 < nchunk)
        def _():
            b = lax.rem(j + 2, NBUF)
            i = lax.rem(j + 2, nis)
            pltpu.make_async_copy(idx_hbm.at[base + j + 2], ib.at[i],
                                  isem.at[i]).wait()
            pltpu.async_copy(y_hbm.at[ib.at[i, 0]], fr.at[b], gsem.at[b])

        b = lax.rem(j, NBUF)
        i = lax.rem(j, nis)
        pltpu.make_async_copy(y_hbm.at[ib.at[i, 0]], fr.at[b],
                              gsem.at[b]).wait()
        pltpu.async_copy(fr.at[b], acc_sh.at[ib.at[i, 1]], ssem.at[b],
                         add=True)
        return c

    lax.fori_loop(0, nchunk, chunk_body, 0)
    j = nchunk - 1
    pltpu.make_async_copy(fr.at[lax.rem(j, NBUF)],
                          acc_sh.at[ib.at[lax.rem(j, nis), 1]],
                          ssem.at[lax.rem(j, NBUF)]).wait()
    plsc.subcore_barrier()
    orow = n_acc // NS
    pltpu.sync_copy(acc_sh.at[pl.ds(sid * orow, orow)],
                    out_hbm.at[cid, pl.ds(sid * orow, orow)])


def _sc_aggregate(y, idx4, n0, n1, n_acc):
    d = y.shape[1]
    mesh = plsc.VectorSubcoreMesh(core_axis_name="c", subcore_axis_name="s")
    f = pl.kernel(
        functools.partial(_agg_body, n0, n1),
        out_type=jax.ShapeDtypeStruct((NC, n_acc, d), jnp.float32),
        mesh=mesh,
        scratch_types=[
            pltpu.VMEM((NISLOT, 2, CHUNK), jnp.int32),
            pltpu.VMEM((NBUF, CHUNK, d), jnp.float32),
            pltpu.VMEM_SHARED((n_acc, d), jnp.float32),
            pltpu.SemaphoreType.DMA((NISLOT,)),
            pltpu.SemaphoreType.DMA((NBUF,)),
            pltpu.SemaphoreType.DMA((NBUF,)),
        ],
    )
    return f(y, idx4)


def _prep_tc(hist, x, w1):
    n, d = x.shape
    g = pl.cdiv(n, N_BLK)

    def body(hist_ref, x_ref, w_ref, y_ref, dis_ref):
        deg = jnp.sum(hist_ref[...], axis=0) + 1.0
        dis = lax.rsqrt(deg)
        y_ref[...] = jnp.dot(x_ref[...], w_ref[...],
                             preferred_element_type=jnp.float32) * dis[:, None]
        dis_ref[...] = dis[:, None]

    return pl.pallas_call(
        body,
        grid=(g,),
        in_specs=[
            pl.BlockSpec((NW, N_BLK), lambda i: (0, i)),
            pl.BlockSpec((N_BLK, d), lambda i: (i, 0)),
            pl.BlockSpec((d, d), lambda i: (0, 0)),
        ],
        out_specs=[
            pl.BlockSpec((N_BLK, d), lambda i: (i, 0)),
            pl.BlockSpec((N_BLK, 1), lambda i: (i, 0)),
        ],
        out_shape=[
            jax.ShapeDtypeStruct((n, d), jnp.float32),
            jax.ShapeDtypeStruct((n, 1), jnp.float32),
        ],
    )(hist, x, w1)


def _mid_tc(p, y, dis, b, w_next):
    n, d = y.shape
    g = pl.cdiv(n, N_BLK)

    def body(p0_ref, p1_ref, y_ref, dis_ref, b_ref, w_ref, o_ref):
        t = p0_ref[0] + p1_ref[0] + y_ref[...]
        h = jnp.maximum(t * dis_ref[...] + b_ref[...], 0.0)
        o_ref[...] = jnp.dot(h, w_ref[...],
                             preferred_element_type=jnp.float32) * dis_ref[...]

    return pl.pallas_call(
        body,
        grid=(g,),
        in_specs=[
            pl.BlockSpec((1, N_BLK, d), lambda i: (0, i, 0)),
            pl.BlockSpec((1, N_BLK, d), lambda i: (1, i, 0)),
            pl.BlockSpec((N_BLK, d), lambda i: (i, 0)),
            pl.BlockSpec((N_BLK, 1), lambda i: (i, 0)),
            pl.BlockSpec((1, d), lambda i: (0, 0)),
            pl.BlockSpec((d, d), lambda i: (0, 0)),
        ],
        out_specs=pl.BlockSpec((N_BLK, d), lambda i: (i, 0)),
        out_shape=jax.ShapeDtypeStruct((n, d), jnp.float32),
    )(p, p, y, dis, b.reshape(1, d), w_next)


def _final_tc(p, y, dis, b):
    n, d = y.shape
    g = pl.cdiv(n, N_BLK)

    def body(p0_ref, p1_ref, y_ref, dis_ref, b_ref, o_ref):
        t = p0_ref[0] + p1_ref[0] + y_ref[...]
        o_ref[...] = jnp.maximum(t * dis_ref[...] + b_ref[...], 0.0)

    return pl.pallas_call(
        body,
        grid=(g,),
        in_specs=[
            pl.BlockSpec((1, N_BLK, d), lambda i: (0, i, 0)),
            pl.BlockSpec((1, N_BLK, d), lambda i: (1, i, 0)),
            pl.BlockSpec((N_BLK, d), lambda i: (i, 0)),
            pl.BlockSpec((N_BLK, 1), lambda i: (i, 0)),
            pl.BlockSpec((1, d), lambda i: (0, 0)),
        ],
        out_specs=pl.BlockSpec((N_BLK, d), lambda i: (i, 0)),
        out_shape=jax.ShapeDtypeStruct((n, d), jnp.float32),
    )(p, p, y, dis, b.reshape(1, d))


def kernel(x, edge_index, W1, b1, W2, b2, W3, b3):
    n, d = x.shape
    e = edge_index.shape[1]
    ei = edge_index.astype(jnp.int32)
    per_pair = (-(-e // CHUNK) + NS - 1) // NS
    if per_pair % 2:
        per_pair += 1
    n0 = int(round(per_pair * 0.90))
    n1 = per_pair - n0
    tot = NS * (n0 + n1)
    e_pad = tot * CHUNK
    pad = e_pad - e
    dummy = n + (jnp.arange(pad, dtype=jnp.int32) % 112)
    src_p = jnp.concatenate([ei[0], jnp.zeros((pad,), jnp.int32)])
    dst_p = jnp.concatenate([ei[1], dummy])
    idx4 = jnp.stack([src_p.reshape(tot, CHUNK),
                      dst_p.reshape(tot, CHUNK)], axis=1)

    n_hist = n + 144
    hist = _sc_hist(idx4, n_hist)
    y, dis = _prep_tc(hist[:, :n], x, W1)

    n_acc = -(-n // (NS * 8)) * NS * 8
    p = _sc_aggregate(y, idx4, n0, n1, n_acc)
    y = _mid_tc(p, y, dis, b1, W2)
    p = _sc_aggregate(y, idx4, n0, n1, n_acc)
    y = _mid_tc(p, y, dis, b2, W3)
    p = _sc_aggregate(y, idx4, n0, n1, n_acc)
    return _final_tc(p, y, dis, b3)


# speedup vs baseline: 1.5351x; 1.0877x over previous
import functools

import jax
import numpy as np
import jax.numpy as jnp
from jax import lax
from jax.experimental import pallas as pl
from jax.experimental.pallas import tpu as pltpu
from jax.experimental.pallas import tpu_sc as plsc

NC = 2
NS = 16
NW = NC * NS
CHUNK = 64
NBUF = 5
GA = NBUF - 1
NISLOT = NBUF + 1
N_BLK = 1024


def _hist_body(idx_hbm, out_hbm, idx_v, hist_v):
    cid = lax.axis_index("c")
    sid = lax.axis_index("s")
    wid = sid * NC + cid
    hc = idx_v.shape[0]
    pltpu.sync_copy(idx_hbm.at[pl.ds(wid * hc, hc)], idx_v)
    nvec = hist_v.shape[0] // 16

    def zero_body(i, c):
        hist_v[pl.ds(i * 16, 16)] = jnp.zeros((16,), jnp.float32)
        return c

    lax.fori_loop(0, nvec, zero_body, 0)
    ones = jnp.ones((16,), jnp.float32)
    nchunk = idx_v.shape[0]

    def chunk_body(j, c):
        for k in range(CHUNK // 16):
            idx = idx_v[j, 1, pl.ds(k * 16, 16)]
            plsc.addupdate_scatter(hist_v, [idx], ones)
        return c

    lax.fori_loop(0, nchunk, chunk_body, 0)
    pltpu.sync_copy(hist_v, out_hbm.at[wid])


def _sc_hist(idx4, n_hist):
    hc = idx4.shape[0] // NW
    mesh = plsc.VectorSubcoreMesh(core_axis_name="c", subcore_axis_name="s")
    f = pl.kernel(
        _hist_body,
        out_type=jax.ShapeDtypeStruct((NW, n_hist), jnp.float32),
        mesh=mesh,
        compiler_params=pltpu.CompilerParams(needs_layout_passes=False),
        scratch_types=[
            pltpu.VMEM((hc, 2, CHUNK), jnp.int32),
            pltpu.VMEM((n_hist,), jnp.float32),
        ],
    )
    return f(idx4)


def _agg_body(n0, n1, y_hbm, idx_hbm, out_hbm,
              ib, fr, acc_sh, isem, gsem, ssem):
    cid = lax.axis_index("c")
    sid = lax.axis_index("s")
    n_acc = acc_sh.shape[0]
    d = acc_sh.shape[1]
    nis = ib.shape[0]
    nchunk = jnp.where(cid == 0, n0, n1)
    base = jnp.where(cid == 0, sid * n0, NS * n0 + sid * n1)

    for i in range(GA + 1):
        pltpu.async_copy(idx_hbm.at[base + i], ib.at[i], isem.at[i])

    def zstore(r, c):
        for k in range(d // 16):
            fr[0, r, pl.ds(k * 16, 16)] = jnp.zeros((16,), jnp.float32)
        return c

    lax.fori_loop(0, CHUNK, zstore, 0)
    zrow = n_acc // NS
    nfull = zrow // CHUNK
    rem = zrow - nfull * CHUNK
    for t in range(nfull):
        pltpu.sync_copy(fr.at[0],
                        acc_sh.at[pl.ds(sid * zrow + t * CHUNK, CHUNK)])
    if rem:
        pltpu.sync_copy(fr.at[0, pl.ds(0, rem)],
                        acc_sh.at[pl.ds(sid * zrow + nfull * CHUNK, rem)])
    plsc.subcore_barrier()

    for i in range(GA):
        pltpu.make_async_copy(idx_hbm.at[base + i], ib.at[i], isem.at[i]).wait()
        pltpu.async_copy(y_hbm.at[ib.at[i, 0]], fr.at[i], gsem.at[i])

    def chunk_body(j, c):
        @pl.when(j >= 1)
        def _():
            b = lax.rem(j - 1, NBUF)
            i = lax.rem(j - 1, nis)
            pltpu.make_async_copy(fr.at[b], acc_sh.at[ib.at[i, 1]],
                                  ssem.at[b]).wait()

        @pl.when(j + GA + 1 < nchunk)
        def _():
            i = lax.rem(j + GA + 1, nis)
            pltpu.async_copy(idx_hbm.at[base + j + GA + 1], ib.at[i],
                             isem.at[i])

        @pl.when(j + GA < nchunk)
        def _():
            b = lax.rem(j + GA, NBUF)
            i = lax.rem(j + GA, nis)
            pltpu.make_async_copy(idx_hbm.at[base + j + GA], ib.at[i],
                                  isem.at[i]).wait()
            pltpu.async_copy(y_hbm.at[ib.at[i, 0]], fr.at[b], gsem.at[b])

        b = lax.rem(j, NBUF)
        i = lax.rem(j, nis)
        pltpu.make_async_copy(y_hbm.at[ib.at[i, 0]], fr.at[b],
                              gsem.at[b]).wait()
        pltpu.async_copy(fr.at[b], acc_sh.at[ib.at[i, 1]], ssem.at[b],
                         add=True)
        return c

    lax.fori_loop(0, nchunk, chunk_body, 0)
    j = nchunk - 1
    pltpu.make_async_copy(fr.at[lax.rem(j, NBUF)],
                          acc_sh.at[ib.at[lax.rem(j, nis), 1]],
                          ssem.at[lax.rem(j, NBUF)]).wait()
    plsc.subcore_barrier()
    orow = n_acc // NS
    pltpu.sync_copy(acc_sh.at[pl.ds(sid * orow, orow)],
                    out_hbm.at[cid, pl.ds(sid * orow, orow)])


def _sc_aggregate(y, idx4, n0, n1, n_acc):
    d = y.shape[1]
    mesh = plsc.VectorSubcoreMesh(core_axis_name="c", subcore_axis_name="s")
    f = pl.kernel(
        functools.partial(_agg_body, n0, n1),
        out_type=jax.ShapeDtypeStruct((NC, n_acc, d), jnp.float32),
        mesh=mesh,
        scratch_types=[
            pltpu.VMEM((NISLOT, 2, CHUNK), jnp.int32),
            pltpu.VMEM((NBUF, CHUNK, d), jnp.float32),
            pltpu.VMEM_SHARED((n_acc, d), jnp.float32),
            pltpu.SemaphoreType.DMA((NISLOT,)),
            pltpu.SemaphoreType.DMA((NBUF,)),
            pltpu.SemaphoreType.DMA((NBUF,)),
        ],
    )
    return f(y, idx4)


def _prep_tc(hist, x, w1):
    n, d = x.shape
    g = pl.cdiv(n, N_BLK)

    def body(hist_ref, x_ref, w_ref, y_ref, dis_ref):
        deg = jnp.sum(hist_ref[...], axis=0) + 1.0
        dis = lax.rsqrt(deg)
        y_ref[...] = jnp.dot(x_ref[...], w_ref[...],
                             preferred_element_type=jnp.float32) * dis[:, None]
        dis_ref[...] = dis[:, None]

    return pl.pallas_call(
        body,
        grid=(g,),
        in_specs=[
            pl.BlockSpec((NW, N_BLK), lambda i: (0, i)),
            pl.BlockSpec((N_BLK, d), lambda i: (i, 0)),
            pl.BlockSpec((d, d), lambda i: (0, 0)),
        ],
        out_specs=[
            pl.BlockSpec((N_BLK, d), lambda i: (i, 0)),
            pl.BlockSpec((N_BLK, 1), lambda i: (i, 0)),
        ],
        out_shape=[
            jax.ShapeDtypeStruct((n, d), jnp.float32),
            jax.ShapeDtypeStruct((n, 1), jnp.float32),
        ],
    )(hist, x, w1)


def _mid_tc(p, y, dis, b, w_next):
    n, d = y.shape
    g = pl.cdiv(n, N_BLK)

    def body(p0_ref, p1_ref, y_ref, dis_ref, b_ref, w_ref, o_ref):
        t = p0_ref[0] + p1_ref[0] + y_ref[...]
        h = jnp.maximum(t * dis_ref[...] + b_ref[...], 0.0)
        o_ref[...] = jnp.dot(h, w_ref[...],
                             preferred_element_type=jnp.float32) * dis_ref[...]

    return pl.pallas_call(
        body,
        grid=(g,),
        in_specs=[
            pl.BlockSpec((1, N_BLK, d), lambda i: (0, i, 0)),
            pl.BlockSpec((1, N_BLK, d), lambda i: (1, i, 0)),
            pl.BlockSpec((N_BLK, d), lambda i: (i, 0)),
            pl.BlockSpec((N_BLK, 1), lambda i: (i, 0)),
            pl.BlockSpec((1, d), lambda i: (0, 0)),
            pl.BlockSpec((d, d), lambda i: (0, 0)),
        ],
        out_specs=pl.BlockSpec((N_BLK, d), lambda i: (i, 0)),
        out_shape=jax.ShapeDtypeStruct((n, d), jnp.float32),
    )(p, p, y, dis, b.reshape(1, d), w_next)


def _final_tc(p, y, dis, b):
    n, d = y.shape
    g = pl.cdiv(n, N_BLK)

    def body(p0_ref, p1_ref, y_ref, dis_ref, b_ref, o_ref):
        t = p0_ref[0] + p1_ref[0] + y_ref[...]
        o_ref[...] = jnp.maximum(t * dis_ref[...] + b_ref[...], 0.0)

    return pl.pallas_call(
        body,
        grid=(g,),
        in_specs=[
            pl.BlockSpec((1, N_BLK, d), lambda i: (0, i, 0)),
            pl.BlockSpec((1, N_BLK, d), lambda i: (1, i, 0)),
            pl.BlockSpec((N_BLK, d), lambda i: (i, 0)),
            pl.BlockSpec((N_BLK, 1), lambda i: (i, 0)),
            pl.BlockSpec((1, d), lambda i: (0, 0)),
        ],
        out_specs=pl.BlockSpec((N_BLK, d), lambda i: (i, 0)),
        out_shape=jax.ShapeDtypeStruct((n, d), jnp.float32),
    )(p, p, y, dis, b.reshape(1, d))


def kernel(x, edge_index, W1, b1, W2, b2, W3, b3):
    n, d = x.shape
    e = edge_index.shape[1]
    ei = edge_index.astype(jnp.int32)
    per_pair = (-(-e // CHUNK) + NS - 1) // NS
    if per_pair % 2:
        per_pair += 1
    n0 = int(round(per_pair * 0.90))
    n1 = per_pair - n0
    tot = NS * (n0 + n1)
    e_pad = tot * CHUNK
    pad = e_pad - e
    dummy = n + (jnp.arange(pad, dtype=jnp.int32) % 112)
    src_p = jnp.concatenate([ei[0], jnp.zeros((pad,), jnp.int32)])
    dst_p = jnp.concatenate([ei[1], dummy])
    idx4 = jnp.stack([src_p.reshape(tot, CHUNK),
                      dst_p.reshape(tot, CHUNK)], axis=1)

    n_hist = n + 144
    hist = _sc_hist(idx4, n_hist)
    y, dis = _prep_tc(hist[:, :n], x, W1)

    n_acc = -(-n // (NS * 8)) * NS * 8
    p = _sc_aggregate(y, idx4, n0, n1, n_acc)
    y = _mid_tc(p, y, dis, b1, W2)
    p = _sc_aggregate(y, idx4, n0, n1, n_acc)
    y = _mid_tc(p, y, dis, b2, W3)
    p = _sc_aggregate(y, idx4, n0, n1, n_acc)
    return _final_tc(p, y, dis, b3)


# speedup vs baseline: 1.6057x; 1.0460x over previous
import functools

import jax
import numpy as np
import jax.numpy as jnp
from jax import lax
from jax.experimental import pallas as pl
from jax.experimental.pallas import tpu as pltpu
from jax.experimental.pallas import tpu_sc as plsc

NC = 2
NS = 16
NW = NC * NS
CHUNK = 64
NBUF = 5
GA = NBUF - 1
NISLOT = NBUF + 1
N_BLK = 1024


def _hist_body(idx_hbm, out_hbm, idx_v, hist_v):
    cid = lax.axis_index("c")
    sid = lax.axis_index("s")
    wid = sid * NC + cid
    hc = idx_v.shape[0]
    pltpu.sync_copy(idx_hbm.at[pl.ds(wid * hc, hc)], idx_v)
    nvec = hist_v.shape[0] // 16

    def zero_body(i, c):
        hist_v[pl.ds(i * 16, 16)] = jnp.zeros((16,), jnp.float32)
        return c

    lax.fori_loop(0, nvec, zero_body, 0)
    ones = jnp.ones((16,), jnp.float32)
    nchunk = idx_v.shape[0]

    def chunk_body(j, c):
        for k in range(CHUNK // 16):
            idx = idx_v[j, 1, pl.ds(k * 16, 16)]
            plsc.addupdate_scatter(hist_v, [idx], ones)
        return c

    lax.fori_loop(0, nchunk, chunk_body, 0)
    pltpu.sync_copy(hist_v, out_hbm.at[wid])


def _sc_hist(idx4, n_hist):
    hc = idx4.shape[0] // NW
    mesh = plsc.VectorSubcoreMesh(core_axis_name="c", subcore_axis_name="s")
    f = pl.kernel(
        _hist_body,
        out_type=jax.ShapeDtypeStruct((NW, n_hist), jnp.float32),
        mesh=mesh,
        compiler_params=pltpu.CompilerParams(needs_layout_passes=False),
        scratch_types=[
            pltpu.VMEM((hc, 2, CHUNK), jnp.int32),
            pltpu.VMEM((n_hist,), jnp.float32),
        ],
    )
    return f(idx4)


def _agg_body(n0, n1, y_hbm, idx_hbm, out_hbm,
              ib, fr, acc_sh, isem, gsem, ssem):
    cid = lax.axis_index("c")
    sid = lax.axis_index("s")
    n_acc = acc_sh.shape[0]
    d = acc_sh.shape[1]
    nis = ib.shape[0]
    nchunk = jnp.where(cid == 0, n0, n1)
    base = jnp.where(cid == 0, sid * n0, NS * n0 + sid * n1)

    for i in range(GA + 1):
        pltpu.async_copy(idx_hbm.at[base + i], ib.at[i], isem.at[i])

    def zstore(r, c):
        for k in range(d // 16):
            fr[0, r, pl.ds(k * 16, 16)] = jnp.zeros((16,), jnp.float32)
        return c

    lax.fori_loop(0, CHUNK, zstore, 0)
    zrow = n_acc // NS
    nfull = zrow // CHUNK
    rem = zrow - nfull * CHUNK
    for t in range(nfull):
        pltpu.sync_copy(fr.at[0],
                        acc_sh.at[pl.ds(sid * zrow + t * CHUNK, CHUNK)])
    if rem:
        pltpu.sync_copy(fr.at[0, pl.ds(0, rem)],
                        acc_sh.at[pl.ds(sid * zrow + nfull * CHUNK, rem)])
    plsc.subcore_barrier()

    for i in range(GA):
        pltpu.make_async_copy(idx_hbm.at[base + i], ib.at[i], isem.at[i]).wait()
        pltpu.async_copy(y_hbm.at[ib.at[i, 0]], fr.at[i], gsem.at[i])

    def chunk_body(j, c):
        @pl.when(j >= 1)
        def _():
            b = lax.rem(j - 1, NBUF)
            i = lax.rem(j - 1, nis)
            pltpu.make_async_copy(fr.at[b], acc_sh.at[ib.at[i, 1]],
                                  ssem.at[b]).wait()

        @pl.when(j + GA + 1 < nchunk)
        def _():
            i = lax.rem(j + GA + 1, nis)
            pltpu.async_copy(idx_hbm.at[base + j + GA + 1], ib.at[i],
                             isem.at[i])

        @pl.when(j + GA < nchunk)
        def _():
            b = lax.rem(j + GA, NBUF)
            i = lax.rem(j + GA, nis)
            pltpu.make_async_copy(idx_hbm.at[base + j + GA], ib.at[i],
                                  isem.at[i]).wait()
            pltpu.async_copy(y_hbm.at[ib.at[i, 0]], fr.at[b], gsem.at[b])

        b = lax.rem(j, NBUF)
        i = lax.rem(j, nis)
        pltpu.make_async_copy(y_hbm.at[ib.at[i, 0]], fr.at[b],
                              gsem.at[b]).wait()
        pltpu.async_copy(fr.at[b], acc_sh.at[ib.at[i, 1]], ssem.at[b],
                         add=True)
        return c

    lax.fori_loop(0, nchunk, chunk_body, 0)
    j = nchunk - 1
    pltpu.make_async_copy(fr.at[lax.rem(j, NBUF)],
                          acc_sh.at[ib.at[lax.rem(j, nis), 1]],
                          ssem.at[lax.rem(j, NBUF)]).wait()
    plsc.subcore_barrier()
    orow = n_acc // NS
    pltpu.sync_copy(acc_sh.at[pl.ds(sid * orow, orow)],
                    out_hbm.at[cid, pl.ds(sid * orow, orow)])


def _sc_aggregate(y, idx4, n0, n1, n_acc):
    d = y.shape[1]
    mesh = plsc.VectorSubcoreMesh(core_axis_name="c", subcore_axis_name="s")
    f = pl.kernel(
        functools.partial(_agg_body, n0, n1),
        out_type=jax.ShapeDtypeStruct((NC, n_acc, d), jnp.float32),
        mesh=mesh,
        scratch_types=[
            pltpu.VMEM((NISLOT, 2, CHUNK), jnp.int32),
            pltpu.VMEM((NBUF, CHUNK, d), jnp.float32),
            pltpu.VMEM_SHARED((n_acc, d), jnp.float32),
            pltpu.SemaphoreType.DMA((NISLOT,)),
            pltpu.SemaphoreType.DMA((NBUF,)),
            pltpu.SemaphoreType.DMA((NBUF,)),
        ],
    )
    return f(y, idx4)


def _prep_tc(hist, x, w1):
    n, d = x.shape
    g = pl.cdiv(n, N_BLK)

    def body(hist_ref, x_ref, w_ref, y_ref, dis_ref):
        deg = jnp.sum(hist_ref[...], axis=0) + 1.0
        dis = lax.rsqrt(deg)
        y_ref[...] = jnp.dot(x_ref[...], w_ref[...],
                             preferred_element_type=jnp.float32) * dis[:, None]
        dis_ref[...] = dis[:, None]

    return pl.pallas_call(
        body,
        grid=(g,),
        in_specs=[
            pl.BlockSpec((NW, N_BLK), lambda i: (0, i)),
            pl.BlockSpec((N_BLK, d), lambda i: (i, 0)),
            pl.BlockSpec((d, d), lambda i: (0, 0)),
        ],
        out_specs=[
            pl.BlockSpec((N_BLK, d), lambda i: (i, 0)),
            pl.BlockSpec((N_BLK, 1), lambda i: (i, 0)),
        ],
        out_shape=[
            jax.ShapeDtypeStruct((n, d), jnp.float32),
            jax.ShapeDtypeStruct((n, 1), jnp.float32),
        ],
    )(hist, x, w1)


def _mid_tc(p, y, dis, b, w_next):
    n, d = y.shape
    g = pl.cdiv(n, N_BLK)

    def body(p0_ref, p1_ref, y_ref, dis_ref, b_ref, w_ref, o_ref):
        t = p0_ref[0] + p1_ref[0] + y_ref[...]
        h = jnp.maximum(t * dis_ref[...] + b_ref[...], 0.0)
        o_ref[...] = jnp.dot(h, w_ref[...],
                             preferred_element_type=jnp.float32) * dis_ref[...]

    return pl.pallas_call(
        body,
        grid=(g,),
        in_specs=[
            pl.BlockSpec((1, N_BLK, d), lambda i: (0, i, 0)),
            pl.BlockSpec((1, N_BLK, d), lambda i: (1, i, 0)),
            pl.BlockSpec((N_BLK, d), lambda i: (i, 0)),
            pl.BlockSpec((N_BLK, 1), lambda i: (i, 0)),
            pl.BlockSpec((1, d), lambda i: (0, 0)),
            pl.BlockSpec((d, d), lambda i: (0, 0)),
        ],
        out_specs=pl.BlockSpec((N_BLK, d), lambda i: (i, 0)),
        out_shape=jax.ShapeDtypeStruct((n, d), jnp.float32),
    )(p, p, y, dis, b.reshape(1, d), w_next)


def _final_tc(p, y, dis, b):
    n, d = y.shape
    g = pl.cdiv(n, N_BLK)

    def body(p0_ref, p1_ref, y_ref, dis_ref, b_ref, o_ref):
        t = p0_ref[0] + p1_ref[0] + y_ref[...]
        o_ref[...] = jnp.maximum(t * dis_ref[...] + b_ref[...], 0.0)

    return pl.pallas_call(
        body,
        grid=(g,),
        in_specs=[
            pl.BlockSpec((1, N_BLK, d), lambda i: (0, i, 0)),
            pl.BlockSpec((1, N_BLK, d), lambda i: (1, i, 0)),
            pl.BlockSpec((N_BLK, d), lambda i: (i, 0)),
            pl.BlockSpec((N_BLK, 1), lambda i: (i, 0)),
            pl.BlockSpec((1, d), lambda i: (0, 0)),
        ],
        out_specs=pl.BlockSpec((N_BLK, d), lambda i: (i, 0)),
        out_shape=jax.ShapeDtypeStruct((n, d), jnp.float32),
    )(p, p, y, dis, b.reshape(1, d))


def kernel(x, edge_index, W1, b1, W2, b2, W3, b3):
    n, d = x.shape
    e = edge_index.shape[1]
    ei = edge_index.astype(jnp.int32)
    per_pair = (-(-e // CHUNK) + NS - 1) // NS
    if per_pair % 2:
        per_pair += 1
    n0 = int(round(per_pair * 0.84))
    n1 = per_pair - n0
    tot = NS * (n0 + n1)
    e_pad = tot * CHUNK
    pad = e_pad - e
    dummy = n + (jnp.arange(pad, dtype=jnp.int32) % 112)
    src_p = jnp.concatenate([ei[0], jnp.zeros((pad,), jnp.int32)])
    dst_p = jnp.concatenate([ei[1], dummy])
    idx4 = jnp.stack([src_p.reshape(tot, CHUNK),
                      dst_p.reshape(tot, CHUNK)], axis=1)

    n_hist = n + 144
    hist = _sc_hist(idx4, n_hist)
    y, dis = _prep_tc(hist[:, :n], x, W1)

    n_acc = -(-n // (NS * 8)) * NS * 8
    p = _sc_aggregate(y, idx4, n0, n1, n_acc)
    y = _mid_tc(p, y, dis, b1, W2)
    p = _sc_aggregate(y, idx4, n0, n1, n_acc)
    y = _mid_tc(p, y, dis, b2, W3)
    p = _sc_aggregate(y, idx4, n0, n1, n_acc)
    return _final_tc(p, y, dis, b3)


# speedup vs baseline: 1.6791x; 1.0457x over previous
import functools

import jax
import numpy as np
import jax.numpy as jnp
from jax import lax
from jax.experimental import pallas as pl
from jax.experimental.pallas import tpu as pltpu
from jax.experimental.pallas import tpu_sc as plsc

NC = 2
NS = 16
NW = NC * NS
CHUNK = 64
NBUF = 5
GA = NBUF - 1
NISLOT = NBUF + 1
N_BLK = 1024


def _hist_body(idx_hbm, out_hbm, idx_v, hist_v):
    cid = lax.axis_index("c")
    sid = lax.axis_index("s")
    wid = sid * NC + cid
    hc = idx_v.shape[0]
    pltpu.sync_copy(idx_hbm.at[pl.ds(wid * hc, hc)], idx_v)
    nvec = hist_v.shape[0] // 16

    def zero_body(i, c):
        hist_v[pl.ds(i * 16, 16)] = jnp.zeros((16,), jnp.float32)
        return c

    lax.fori_loop(0, nvec, zero_body, 0)
    ones = jnp.ones((16,), jnp.float32)
    nchunk = idx_v.shape[0]

    def chunk_body(j, c):
        for k in range(CHUNK // 16):
            idx = idx_v[j, 1, pl.ds(k * 16, 16)]
            plsc.addupdate_scatter(hist_v, [idx], ones)
        return c

    lax.fori_loop(0, nchunk, chunk_body, 0)
    pltpu.sync_copy(hist_v, out_hbm.at[wid])


def _sc_hist(idx4, n_hist):
    hc = idx4.shape[0] // NW
    mesh = plsc.VectorSubcoreMesh(core_axis_name="c", subcore_axis_name="s")
    f = pl.kernel(
        _hist_body,
        out_type=jax.ShapeDtypeStruct((NW, n_hist), jnp.float32),
        mesh=mesh,
        compiler_params=pltpu.CompilerParams(needs_layout_passes=False),
        scratch_types=[
            pltpu.VMEM((hc, 2, CHUNK), jnp.int32),
            pltpu.VMEM((n_hist,), jnp.float32),
        ],
    )
    return f(idx4)


def _agg_body(n0, n1, y_hbm, idx_hbm, out_hbm,
              ib, fr, acc_sh, isem, gsem, ssem):
    cid = lax.axis_index("c")
    sid = lax.axis_index("s")
    n_acc = acc_sh.shape[0]
    d = acc_sh.shape[1]
    nis = ib.shape[0]
    nchunk = jnp.where(cid == 0, n0, n1)
    base = jnp.where(cid == 0, sid * n0, NS * n0 + sid * n1)

    for i in range(GA + 1):
        pltpu.async_copy(idx_hbm.at[base + i], ib.at[i], isem.at[i])

    def zstore(r, c):
        for k in range(d // 16):
            fr[0, r, pl.ds(k * 16, 16)] = jnp.zeros((16,), jnp.float32)
        return c

    lax.fori_loop(0, CHUNK, zstore, 0)
    zrow = n_acc // NS
    nfull = zrow // CHUNK
    rem = zrow - nfull * CHUNK
    for t in range(nfull):
        pltpu.sync_copy(fr.at[0],
                        acc_sh.at[pl.ds(sid * zrow + t * CHUNK, CHUNK)])
    if rem:
        pltpu.sync_copy(fr.at[0, pl.ds(0, rem)],
                        acc_sh.at[pl.ds(sid * zrow + nfull * CHUNK, rem)])
    plsc.subcore_barrier()

    for i in range(GA):
        pltpu.make_async_copy(idx_hbm.at[base + i], ib.at[i], isem.at[i]).wait()
        pltpu.async_copy(y_hbm.at[ib.at[i, 0]], fr.at[i], gsem.at[i])

    def chunk_body(j, c):
        @pl.when(j >= 1)
        def _():
            b = lax.rem(j - 1, NBUF)
            i = lax.rem(j - 1, nis)
            pltpu.make_async_copy(fr.at[b], acc_sh.at[ib.at[i, 1]],
                                  ssem.at[b]).wait()

        @pl.when(j + GA + 1 < nchunk)
        def _():
            i = lax.rem(j + GA + 1, nis)
            pltpu.async_copy(idx_hbm.at[base + j + GA + 1], ib.at[i],
                             isem.at[i])

        @pl.when(j + GA < nchunk)
        def _():
            b = lax.rem(j + GA, NBUF)
            i = lax.rem(j + GA, nis)
            pltpu.make_async_copy(idx_hbm.at[base + j + GA], ib.at[i],
                                  isem.at[i]).wait()
            pltpu.async_copy(y_hbm.at[ib.at[i, 0]], fr.at[b], gsem.at[b])

        b = lax.rem(j, NBUF)
        i = lax.rem(j, nis)
        pltpu.make_async_copy(y_hbm.at[ib.at[i, 0]], fr.at[b],
                              gsem.at[b]).wait()
        pltpu.async_copy(fr.at[b], acc_sh.at[ib.at[i, 1]], ssem.at[b],
                         add=True)
        return c

    lax.fori_loop(0, nchunk, chunk_body, 0)
    j = nchunk - 1
    pltpu.make_async_copy(fr.at[lax.rem(j, NBUF)],
                          acc_sh.at[ib.at[lax.rem(j, nis), 1]],
                          ssem.at[lax.rem(j, NBUF)]).wait()
    plsc.subcore_barrier()
    orow = n_acc // NS
    pltpu.sync_copy(acc_sh.at[pl.ds(sid * orow, orow)],
                    out_hbm.at[cid, pl.ds(sid * orow, orow)])


def _sc_aggregate(y, idx4, n0, n1, n_acc):
    d = y.shape[1]
    mesh = plsc.VectorSubcoreMesh(core_axis_name="c", subcore_axis_name="s")
    f = pl.kernel(
        functools.partial(_agg_body, n0, n1),
        out_type=jax.ShapeDtypeStruct((NC, n_acc, d), jnp.float32),
        mesh=mesh,
        scratch_types=[
            pltpu.VMEM((NISLOT, 2, CHUNK), jnp.int32),
            pltpu.VMEM((NBUF, CHUNK, d), jnp.float32),
            pltpu.VMEM_SHARED((n_acc, d), jnp.float32),
            pltpu.SemaphoreType.DMA((NISLOT,)),
            pltpu.SemaphoreType.DMA((NBUF,)),
            pltpu.SemaphoreType.DMA((NBUF,)),
        ],
    )
    return f(y, idx4)


def _prep_tc(hist, x, w1):
    n, d = x.shape
    g = pl.cdiv(n, N_BLK)

    def body(hist_ref, x_ref, w_ref, y_ref, dis_ref):
        deg = jnp.sum(hist_ref[...], axis=0) + 1.0
        dis = lax.rsqrt(deg)
        y_ref[...] = jnp.dot(x_ref[...], w_ref[...],
                             preferred_element_type=jnp.float32) * dis[:, None]
        dis_ref[...] = dis[:, None]

    return pl.pallas_call(
        body,
        grid=(g,),
        in_specs=[
            pl.BlockSpec((NW, N_BLK), lambda i: (0, i)),
            pl.BlockSpec((N_BLK, d), lambda i: (i, 0)),
            pl.BlockSpec((d, d), lambda i: (0, 0)),
        ],
        out_specs=[
            pl.BlockSpec((N_BLK, d), lambda i: (i, 0)),
            pl.BlockSpec((N_BLK, 1), lambda i: (i, 0)),
        ],
        out_shape=[
            jax.ShapeDtypeStruct((n, d), jnp.float32),
            jax.ShapeDtypeStruct((n, 1), jnp.float32),
        ],
    )(hist, x, w1)


def _mid_tc(p, y, dis, b, w_next):
    n, d = y.shape
    g = pl.cdiv(n, N_BLK)

    def body(p0_ref, p1_ref, y_ref, dis_ref, b_ref, w_ref, o_ref):
        t = p0_ref[0] + p1_ref[0] + y_ref[...]
        h = jnp.maximum(t * dis_ref[...] + b_ref[...], 0.0)
        o_ref[...] = jnp.dot(h, w_ref[...],
                             preferred_element_type=jnp.float32) * dis_ref[...]

    return pl.pallas_call(
        body,
        grid=(g,),
        in_specs=[
            pl.BlockSpec((1, N_BLK, d), lambda i: (0, i, 0)),
            pl.BlockSpec((1, N_BLK, d), lambda i: (1, i, 0)),
            pl.BlockSpec((N_BLK, d), lambda i: (i, 0)),
            pl.BlockSpec((N_BLK, 1), lambda i: (i, 0)),
            pl.BlockSpec((1, d), lambda i: (0, 0)),
            pl.BlockSpec((d, d), lambda i: (0, 0)),
        ],
        out_specs=pl.BlockSpec((N_BLK, d), lambda i: (i, 0)),
        out_shape=jax.ShapeDtypeStruct((n, d), jnp.float32),
    )(p, p, y, dis, b.reshape(1, d), w_next)


def _final_tc(p, y, dis, b):
    n, d = y.shape
    g = pl.cdiv(n, N_BLK)

    def body(p0_ref, p1_ref, y_ref, dis_ref, b_ref, o_ref):
        t = p0_ref[0] + p1_ref[0] + y_ref[...]
        o_ref[...] = jnp.maximum(t * dis_ref[...] + b_ref[...], 0.0)

    return pl.pallas_call(
        body,
        grid=(g,),
        in_specs=[
            pl.BlockSpec((1, N_BLK, d), lambda i: (0, i, 0)),
            pl.BlockSpec((1, N_BLK, d), lambda i: (1, i, 0)),
            pl.BlockSpec((N_BLK, d), lambda i: (i, 0)),
            pl.BlockSpec((N_BLK, 1), lambda i: (i, 0)),
            pl.BlockSpec((1, d), lambda i: (0, 0)),
        ],
        out_specs=pl.BlockSpec((N_BLK, d), lambda i: (i, 0)),
        out_shape=jax.ShapeDtypeStruct((n, d), jnp.float32),
    )(p, p, y, dis, b.reshape(1, d))


def kernel(x, edge_index, W1, b1, W2, b2, W3, b3):
    n, d = x.shape
    e = edge_index.shape[1]
    ei = edge_index.astype(jnp.int32)
    per_pair = (-(-e // CHUNK) + NS - 1) // NS
    if per_pair % 2:
        per_pair += 1
    n0 = int(round(per_pair * 0.79))
    n1 = per_pair - n0
    tot = NS * (n0 + n1)
    e_pad = tot * CHUNK
    pad = e_pad - e
    dummy = n + (jnp.arange(pad, dtype=jnp.int32) % 112)
    src_p = jnp.concatenate([ei[0], jnp.zeros((pad,), jnp.int32)])
    dst_p = jnp.concatenate([ei[1], dummy])
    idx4 = jnp.stack([src_p.reshape(tot, CHUNK),
                      dst_p.reshape(tot, CHUNK)], axis=1)

    n_hist = n + 144
    hist = _sc_hist(idx4, n_hist)
    y, dis = _prep_tc(hist[:, :n], x, W1)

    n_acc = -(-n // (NS * 8)) * NS * 8
    p = _sc_aggregate(y, idx4, n0, n1, n_acc)
    y = _mid_tc(p, y, dis, b1, W2)
    p = _sc_aggregate(y, idx4, n0, n1, n_acc)
    y = _mid_tc(p, y, dis, b2, W3)
    p = _sc_aggregate(y, idx4, n0, n1, n_acc)
    return _final_tc(p, y, dis, b3)


# speedup vs baseline: 1.6799x; 1.0005x over previous
import functools

import jax
import numpy as np
import jax.numpy as jnp
from jax import lax
from jax.experimental import pallas as pl
from jax.experimental.pallas import tpu as pltpu
from jax.experimental.pallas import tpu_sc as plsc

NC = 2
NS = 16
NW = NC * NS
CHUNK = 64
NBUF = 5
GA = NBUF - 1
NISLOT = NBUF + 1
N_BLK = 1024


def _hist_body(idx_hbm, out_hbm, idx_v, hist_v):
    cid = lax.axis_index("c")
    sid = lax.axis_index("s")
    wid = sid * NC + cid
    hc = idx_v.shape[0]
    pltpu.sync_copy(idx_hbm.at[pl.ds(wid * hc, hc)], idx_v)
    nvec = hist_v.shape[0] // 16

    def zero_body(i, c):
        hist_v[pl.ds(i * 16, 16)] = jnp.zeros((16,), jnp.float32)
        return c

    lax.fori_loop(0, nvec, zero_body, 0)
    ones = jnp.ones((16,), jnp.float32)
    nchunk = idx_v.shape[0]

    def chunk_body(j, c):
        for k in range(CHUNK // 16):
            idx = idx_v[j, 1, pl.ds(k * 16, 16)]
            plsc.addupdate_scatter(hist_v, [idx], ones)
        return c

    lax.fori_loop(0, nchunk, chunk_body, 0)
    pltpu.sync_copy(hist_v, out_hbm.at[wid])


def _sc_hist(idx4, n_hist):
    hc = idx4.shape[0] // NW
    mesh = plsc.VectorSubcoreMesh(core_axis_name="c", subcore_axis_name="s")
    f = pl.kernel(
        _hist_body,
        out_type=jax.ShapeDtypeStruct((NW, n_hist), jnp.float32),
        mesh=mesh,
        compiler_params=pltpu.CompilerParams(needs_layout_passes=False),
        scratch_types=[
            pltpu.VMEM((hc, 2, CHUNK), jnp.int32),
            pltpu.VMEM((n_hist,), jnp.float32),
        ],
    )
    return f(idx4)


def _agg_body(n0, n1, y_hbm, idx_hbm, out_hbm,
              ib, fr, acc_sh, isem, gsem, ssem):
    cid = lax.axis_index("c")
    sid = lax.axis_index("s")
    n_acc = acc_sh.shape[0]
    d = acc_sh.shape[1]
    nis = ib.shape[0]
    nchunk = jnp.where(cid == 0, n0, n1)
    base = jnp.where(cid == 0, sid * n0, NS * n0 + sid * n1)

    for i in range(GA + 1):
        pltpu.async_copy(idx_hbm.at[base + i], ib.at[i], isem.at[i])

    def zstore(r, c):
        for k in range(d // 16):
            fr[0, r, pl.ds(k * 16, 16)] = jnp.zeros((16,), jnp.float32)
        return c

    lax.fori_loop(0, CHUNK, zstore, 0)
    zrow = n_acc // NS
    nfull = zrow // CHUNK
    rem = zrow - nfull * CHUNK
    for t in range(nfull):
        pltpu.sync_copy(fr.at[0],
                        acc_sh.at[pl.ds(sid * zrow + t * CHUNK, CHUNK)])
    if rem:
        pltpu.sync_copy(fr.at[0, pl.ds(0, rem)],
                        acc_sh.at[pl.ds(sid * zrow + nfull * CHUNK, rem)])
    plsc.subcore_barrier()

    for i in range(GA):
        pltpu.make_async_copy(idx_hbm.at[base + i], ib.at[i], isem.at[i]).wait()
        pltpu.async_copy(y_hbm.at[ib.at[i, 0]], fr.at[i], gsem.at[i])

    def chunk_body(j, c):
        @pl.when(j >= 1)
        def _():
            b = lax.rem(j - 1, NBUF)
            i = lax.rem(j - 1, nis)
            pltpu.make_async_copy(fr.at[b], acc_sh.at[ib.at[i, 1]],
                                  ssem.at[b]).wait()

        @pl.when(j + GA + 1 < nchunk)
        def _():
            i = lax.rem(j + GA + 1, nis)
            pltpu.async_copy(idx_hbm.at[base + j + GA + 1], ib.at[i],
                             isem.at[i])

        @pl.when(j + GA < nchunk)
        def _():
            b = lax.rem(j + GA, NBUF)
            i = lax.rem(j + GA, nis)
            pltpu.make_async_copy(idx_hbm.at[base + j + GA], ib.at[i],
                                  isem.at[i]).wait()
            pltpu.async_copy(y_hbm.at[ib.at[i, 0]], fr.at[b], gsem.at[b])

        b = lax.rem(j, NBUF)
        i = lax.rem(j, nis)
        pltpu.make_async_copy(y_hbm.at[ib.at[i, 0]], fr.at[b],
                              gsem.at[b]).wait()
        pltpu.async_copy(fr.at[b], acc_sh.at[ib.at[i, 1]], ssem.at[b],
                         add=True)
        return c

    lax.fori_loop(0, nchunk, chunk_body, 0)
    j = nchunk - 1
    pltpu.make_async_copy(fr.at[lax.rem(j, NBUF)],
                          acc_sh.at[ib.at[lax.rem(j, nis), 1]],
                          ssem.at[lax.rem(j, NBUF)]).wait()
    plsc.subcore_barrier()
    orow = n_acc // NS
    pltpu.sync_copy(acc_sh.at[pl.ds(sid * orow, orow)],
                    out_hbm.at[cid, pl.ds(sid * orow, orow)])


def _sc_aggregate(y, idx4, n0, n1, n_acc):
    d = y.shape[1]
    mesh = plsc.VectorSubcoreMesh(core_axis_name="c", subcore_axis_name="s")
    f = pl.kernel(
        functools.partial(_agg_body, n0, n1),
        out_type=jax.ShapeDtypeStruct((NC, n_acc, d), jnp.float32),
        mesh=mesh,
        scratch_types=[
            pltpu.VMEM((NISLOT, 2, CHUNK), jnp.int32),
            pltpu.VMEM((NBUF, CHUNK, d), jnp.float32),
            pltpu.VMEM_SHARED((n_acc, d), jnp.float32),
            pltpu.SemaphoreType.DMA((NISLOT,)),
            pltpu.SemaphoreType.DMA((NBUF,)),
            pltpu.SemaphoreType.DMA((NBUF,)),
        ],
    )
    return f(y, idx4)


def _prep_tc(hist, x, w1):
    n, d = x.shape
    g = pl.cdiv(n, N_BLK)

    def body(hist_ref, x_ref, w_ref, y_ref, dis_ref):
        deg = jnp.sum(hist_ref[...], axis=0) + 1.0
        dis = lax.rsqrt(deg)
        y_ref[...] = jnp.dot(x_ref[...], w_ref[...],
                             preferred_element_type=jnp.float32) * dis[:, None]
        dis_ref[...] = dis[:, None]

    return pl.pallas_call(
        body,
        grid=(g,),
        in_specs=[
            pl.BlockSpec((NW, N_BLK), lambda i: (0, i)),
            pl.BlockSpec((N_BLK, d), lambda i: (i, 0)),
            pl.BlockSpec((d, d), lambda i: (0, 0)),
        ],
        out_specs=[
            pl.BlockSpec((N_BLK, d), lambda i: (i, 0)),
            pl.BlockSpec((N_BLK, 1), lambda i: (i, 0)),
        ],
        out_shape=[
            jax.ShapeDtypeStruct((n, d), jnp.float32),
            jax.ShapeDtypeStruct((n, 1), jnp.float32),
        ],
    )(hist, x, w1)


def _mid_tc(p, y, dis, b, w_next):
    n, d = y.shape
    g = pl.cdiv(n, N_BLK)

    def body(p0_ref, p1_ref, y_ref, dis_ref, b_ref, w_ref, o_ref):
        t = p0_ref[0] + p1_ref[0] + y_ref[...]
        h = jnp.maximum(t * dis_ref[...] + b_ref[...], 0.0)
        o_ref[...] = jnp.dot(h, w_ref[...],
                             preferred_element_type=jnp.float32) * dis_ref[...]

    return pl.pallas_call(
        body,
        grid=(g,),
        in_specs=[
            pl.BlockSpec((1, N_BLK, d), lambda i: (0, i, 0)),
            pl.BlockSpec((1, N_BLK, d), lambda i: (1, i, 0)),
            pl.BlockSpec((N_BLK, d), lambda i: (i, 0)),
            pl.BlockSpec((N_BLK, 1), lambda i: (i, 0)),
            pl.BlockSpec((1, d), lambda i: (0, 0)),
            pl.BlockSpec((d, d), lambda i: (0, 0)),
        ],
        out_specs=pl.BlockSpec((N_BLK, d), lambda i: (i, 0)),
        out_shape=jax.ShapeDtypeStruct((n, d), jnp.float32),
    )(p, p, y, dis, b.reshape(1, d), w_next)


def _final_tc(p, y, dis, b):
    n, d = y.shape
    g = pl.cdiv(n, N_BLK)

    def body(p0_ref, p1_ref, y_ref, dis_ref, b_ref, o_ref):
        t = p0_ref[0] + p1_ref[0] + y_ref[...]
        o_ref[...] = jnp.maximum(t * dis_ref[...] + b_ref[...], 0.0)

    return pl.pallas_call(
        body,
        grid=(g,),
        in_specs=[
            pl.BlockSpec((1, N_BLK, d), lambda i: (0, i, 0)),
            pl.BlockSpec((1, N_BLK, d), lambda i: (1, i, 0)),
            pl.BlockSpec((N_BLK, d), lambda i: (i, 0)),
            pl.BlockSpec((N_BLK, 1), lambda i: (i, 0)),
            pl.BlockSpec((1, d), lambda i: (0, 0)),
        ],
        out_specs=pl.BlockSpec((N_BLK, d), lambda i: (i, 0)),
        out_shape=jax.ShapeDtypeStruct((n, d), jnp.float32),
    )(p, p, y, dis, b.reshape(1, d))


def kernel(x, edge_index, W1, b1, W2, b2, W3, b3):
    n, d = x.shape
    e = edge_index.shape[1]
    ei = edge_index.astype(jnp.int32)
    per_pair = (-(-e // CHUNK) + NS - 1) // NS
    if per_pair % 2:
        per_pair += 1
    n0 = int(round(per_pair * 0.74))
    n1 = per_pair - n0
    tot = NS * (n0 + n1)
    e_pad = tot * CHUNK
    pad = e_pad - e
    dummy = n + (jnp.arange(pad, dtype=jnp.int32) % 112)
    src_p = jnp.concatenate([ei[0], jnp.zeros((pad,), jnp.int32)])
    dst_p = jnp.concatenate([ei[1], dummy])
    idx4 = jnp.stack([src_p.reshape(tot, CHUNK),
                      dst_p.reshape(tot, CHUNK)], axis=1)

    n_hist = n + 144
    hist = _sc_hist(idx4, n_hist)
    y, dis = _prep_tc(hist[:, :n], x, W1)

    n_acc = -(-n // (NS * 8)) * NS * 8
    p = _sc_aggregate(y, idx4, n0, n1, n_acc)
    y = _mid_tc(p, y, dis, b1, W2)
    p = _sc_aggregate(y, idx4, n0, n1, n_acc)
    y = _mid_tc(p, y, dis, b2, W3)
    p = _sc_aggregate(y, idx4, n0, n1, n_acc)
    return _final_tc(p, y, dis, b3)
